# Initial kernel scaffold; baseline (speedup 1.0000x reference)
import jax
import jax.numpy as jnp
from jax.experimental import pallas as pl


def kernel(x, edge_index, edge_attr, batch, atom_emb, chir_emb, bond_type_emb, bond_dir_emb, W1, b1, W2, b2, bn_gamma, bn_beta, gate_w, gate_b, pred_w, pred_b):
    raise NotImplementedError("write your pallas kernel here")


# speedup vs baseline: 4.7735x; 4.7735x over previous
import functools

import jax
import jax.numpy as jnp
from jax import lax
from jax.experimental import pallas as pl
from jax.experimental.pallas import tpu as pltpu
from jax.experimental.pallas import tpu_sc as plsc

N = 10000
NPAD = 10240
D = 300
DP = 320
DH = 160
E = 160000
EPAD = 163840
L = 5
G = 256
HID = 640
CNT = 16
NC, NS = 2, 16
CHUNK = 128
RPT = NPAD // NS

_HIGH = lax.Precision.HIGHEST


@functools.cache
def _sc_mesh():
    return plsc.VectorSubcoreMesh(
        core_axis_name="c", subcore_axis_name="s", num_cores=NC,
        num_subcores=NS)


def _neigh_body_fn(hA, hB, src, dst, zrows, out, acc, zbuf, idx_s, idx_d, rows, sem):
    cid = lax.axis_index("c")
    sid = lax.axis_index("s")

    pltpu.sync_copy(zrows, zbuf)

    def zbody(i, carry):
        pltpu.sync_copy(zbuf, acc.at[pl.ds(sid * RPT + i * 16, 16)])
        return carry

    lax.fori_loop(0, RPT // 16, zbody, 0)
    plsc.subcore_barrier()

    ept = EPAD // NS

    def ebody(c, carry):
        b = sid * ept + c * CHUNK
        pltpu.sync_copy(src.at[pl.ds(b, CHUNK)], idx_s)
        pltpu.sync_copy(dst.at[pl.ds(b, CHUNK)], idx_d)

        @pl.when(cid == 0)
        def _():
            pltpu.async_copy(hA.at[idx_s], rows, sem).wait()

        @pl.when(cid == 1)
        def _():
            pltpu.async_copy(hB.at[idx_s], rows, sem).wait()

        pltpu.sync_copy(rows, acc.at[idx_d], add=True)
        return carry

    lax.fori_loop(0, ept // CHUNK, ebody, 0)
    plsc.subcore_barrier()

    def dbody(i, carry):
        r = sid * RPT + i * 16
        pltpu.sync_copy(acc.at[pl.ds(r, 16)], zbuf)
        pltpu.sync_copy(zbuf, out.at[cid].at[pl.ds(r, 16)])
        return carry

    lax.fori_loop(0, RPT // 16, dbody, 0)


@functools.cache
def _build_neigh_kernel():
    return pl.kernel(
        _neigh_body_fn,
        out_type=jax.ShapeDtypeStruct((2, NPAD, DH), jnp.float32),
        mesh=_sc_mesh(),
        scratch_types=[
            pltpu.VMEM_SHARED((NPAD, DH), jnp.float32),
            pltpu.VMEM((16, DH), jnp.float32),
            pltpu.VMEM((CHUNK,), jnp.int32),
            pltpu.VMEM((CHUNK,), jnp.int32),
            pltpu.VMEM((CHUNK, DH), jnp.float32),
            pltpu.SemaphoreType.DMA,
        ],
        compiler_params=pltpu.CompilerParams(use_tc_tiling_on_sc=False, needs_layout_passes=False),
    )


def _neighbor_sum(hA, hB, srcp, dstp, zrows):
    return _build_neigh_kernel()(hA, hB, srcp, dstp, zrows)


def _counts_body_fn(ea0, ea1, dst, zrows, out,
                    acc, zbuf, rowsc, idx_t, idx_r, idx_d):
    cid = lax.axis_index("c")
    sid = lax.axis_index("s")
    wid = sid * NC + cid

    pltpu.sync_copy(zrows, zbuf)

    def zbody(i, carry):
        pltpu.sync_copy(zbuf, acc.at[pl.ds(sid * RPT + i * 64, 64)])
        return carry

    lax.fori_loop(0, RPT // 64, zbody, 0)
    plsc.subcore_barrier()

    epw = EPAD // (NC * NS)
    ones = jnp.ones((16,), jnp.float32)

    def ebody(c, carry):
        b = wid * epw + c * CHUNK
        pltpu.sync_copy(ea0.at[pl.ds(b, CHUNK)], idx_t)
        pltpu.sync_copy(ea1.at[pl.ds(b, CHUNK)], idx_r)
        pltpu.sync_copy(dst.at[pl.ds(b, CHUNK)], idx_d)
        zero16 = jnp.zeros((16,), jnp.float32)
        for i in range(CHUNK):
            rowsc[i, :] = zero16
        for g in range(CHUNK // 16):
            rid = lax.iota(jnp.int32, 16) + g * 16
            t16 = idx_t[pl.ds(g * 16, 16)]
            plsc.addupdate_scatter(rowsc, [rid, t16], ones)
            r16 = idx_r[pl.ds(g * 16, 16)]
            plsc.addupdate_scatter(rowsc, [rid, r16 + 6], ones)
        pltpu.sync_copy(rowsc, acc.at[idx_d], add=True)
        return carry

    lax.fori_loop(0, epw // CHUNK, ebody, 0)
    plsc.subcore_barrier()

    def dbody(i, carry):
        r = sid * RPT + i * 64
        pltpu.sync_copy(acc.at[pl.ds(r, 64)], zbuf)
        pltpu.sync_copy(zbuf, out.at[cid].at[pl.ds(r, 64)])
        return carry

    lax.fori_loop(0, RPT // 64, dbody, 0)


@functools.cache
def _build_counts_kernel():
    return pl.kernel(
        _counts_body_fn,
        out_type=jax.ShapeDtypeStruct((2, NPAD, CNT), jnp.float32),
        mesh=_sc_mesh(),
        scratch_types=[
            pltpu.VMEM_SHARED((NPAD, CNT), jnp.float32),
            pltpu.VMEM((64, CNT), jnp.float32),
            pltpu.VMEM((CHUNK, CNT), jnp.float32),
            pltpu.VMEM((CHUNK,), jnp.int32),
            pltpu.VMEM((CHUNK,), jnp.int32),
            pltpu.VMEM((CHUNK,), jnp.int32),
        ],
        compiler_params=pltpu.CompilerParams(use_tc_tiling_on_sc=False, needs_layout_passes=False),
    )


def _edge_counts(ea0p, ea1p, dstp, zrows):
    return _build_counts_kernel()(ea0p, ea1p, dstp, zrows)


_BR = 1024


def _init_body(x_ref, ae_ref, ce_ref, out_ref):
    xb = x_ref[...]
    a = xb[:, 0:1]
    c = xb[:, 1:2]
    oh_a = (lax.broadcasted_iota(jnp.int32, (_BR, 128), 1) == a).astype(jnp.float32)
    oh_c = (lax.broadcasted_iota(jnp.int32, (_BR, 8), 1) == c).astype(jnp.float32)
    h = jnp.dot(oh_a, ae_ref[...], precision=_HIGH) + jnp.dot(
        oh_c, ce_ref[...], precision=_HIGH)
    out_ref[0, :, :] = h[:, :DH]
    out_ref[1, :, :] = h[:, DH:]


def _init_nodes(x_p, atom_emb_p, chir_emb_p):
    return pl.pallas_call(
        _init_body,
        grid=(NPAD // _BR,),
        in_specs=[
            pl.BlockSpec((_BR, 2), lambda i: (i, 0)),
            pl.BlockSpec((128, DP), lambda i: (0, 0)),
            pl.BlockSpec((8, DP), lambda i: (0, 0)),
        ],
        out_specs=pl.BlockSpec((2, _BR, DH), lambda i: (0, i, 0)),
        out_shape=jax.ShapeDtypeStruct((2, NPAD, DH), jnp.float32),
        compiler_params=pltpu.CompilerParams(
            dimension_semantics=("parallel",)),
    )(x_p, atom_emb_p, chir_emb_p)


def _layer_body(relu, p_ref, h_ref, c_ref, ctab_ref, sl_ref, w1_ref, b1_ref,
                w2_ref, b2_ref, sc_ref, be_ref, out_ref):
    aggA = p_ref[0] + h_ref[0]
    aggB = p_ref[1] + h_ref[1]
    agg = jnp.concatenate([aggA, aggB], axis=1)
    cnt = c_ref[0] + c_ref[1]
    agg = agg + jnp.dot(cnt, ctab_ref[...], precision=_HIGH) + sl_ref[...]
    z = jnp.maximum(jnp.dot(agg, w1_ref[...], precision=_HIGH) + b1_ref[...], 0.0)
    z = jnp.dot(z, w2_ref[...], precision=_HIGH) + b2_ref[...]
    z = z * sc_ref[...] + be_ref[...]
    if relu:
        z = jnp.maximum(z, 0.0)
    out_ref[0, :, :] = z[:, :DH]
    out_ref[1, :, :] = z[:, DH:]


def _gin_layer(P, h, C, ctab, sl, w1, b1, w2, b2, scale, beta, relu):
    return pl.pallas_call(
        functools.partial(_layer_body, relu),
        grid=(NPAD // _BR,),
        in_specs=[
            pl.BlockSpec((2, _BR, DH), lambda i: (0, i, 0)),
            pl.BlockSpec((2, _BR, DH), lambda i: (0, i, 0)),
            pl.BlockSpec((2, _BR, CNT), lambda i: (0, i, 0)),
            pl.BlockSpec((CNT, DP), lambda i: (0, 0)),
            pl.BlockSpec((1, DP), lambda i: (0, 0)),
            pl.BlockSpec((DP, HID), lambda i: (0, 0)),
            pl.BlockSpec((1, HID), lambda i: (0, 0)),
            pl.BlockSpec((HID, DP), lambda i: (0, 0)),
            pl.BlockSpec((1, DP), lambda i: (0, 0)),
            pl.BlockSpec((1, DP), lambda i: (0, 0)),
            pl.BlockSpec((1, DP), lambda i: (0, 0)),
        ],
        out_specs=pl.BlockSpec((2, _BR, DH), lambda i: (0, i, 0)),
        out_shape=jax.ShapeDtypeStruct((2, NPAD, DH), jnp.float32),
        compiler_params=pltpu.CompilerParams(
            dimension_semantics=("parallel",)),
    )(P, h, C, ctab, sl, w1, b1, w2, b2, scale, beta)


def _gate_body(h_ref, b_ref, gw_ref, gate_ref, gmax_ref, acc_ref):
    i = pl.program_id(0)
    h = jnp.concatenate([h_ref[0], h_ref[1]], axis=1)
    gate = jnp.dot(h, gw_ref[...], precision=_HIGH)[:, 0:1]
    gate_ref[...] = gate
    seg = lax.broadcasted_iota(jnp.int32, (_BR, G), 1) == b_ref[...]
    bmax = jnp.max(jnp.where(seg, gate, -jnp.inf), axis=0, keepdims=True)

    @pl.when(i == 0)
    def _():
        acc_ref[...] = jnp.full((8, G), -jnp.inf, jnp.float32)

    acc_ref[0:1, :] = jnp.maximum(acc_ref[0:1, :], bmax)
    gmax_ref[...] = jnp.where(
        jnp.isfinite(acc_ref[0:1, :]), acc_ref[0:1, :], 0.0)


def _gate_sweep(h, batch_p, gate_w_p):
    return pl.pallas_call(
        _gate_body,
        grid=(NPAD // _BR,),
        in_specs=[
            pl.BlockSpec((2, _BR, DH), lambda i: (0, i, 0)),
            pl.BlockSpec((_BR, 1), lambda i: (i, 0)),
            pl.BlockSpec((DP, 128), lambda i: (0, 0)),
        ],
        out_specs=[
            pl.BlockSpec((_BR, 1), lambda i: (i, 0)),
            pl.BlockSpec((1, G), lambda i: (0, 0)),
        ],
        out_shape=[
            jax.ShapeDtypeStruct((NPAD, 1), jnp.float32),
            jax.ShapeDtypeStruct((1, G), jnp.float32),
        ],
        scratch_shapes=[pltpu.VMEM((8, G), jnp.float32)],
        compiler_params=pltpu.CompilerParams(
            dimension_semantics=("arbitrary",)),
    )(h, batch_p, gate_w_p)


def _pool_body(h_ref, g_ref, b_ref, gmax_ref, pw_ref, pb_ref, out_ref,
               den_ref, u_ref):
    i = pl.program_id(0)

    @pl.when(i == 0)
    def _():
        den_ref[...] = jnp.zeros_like(den_ref)
        u_ref[...] = jnp.zeros_like(u_ref)

    h = jnp.concatenate([h_ref[0], h_ref[1]], axis=1)
    b = b_ref[...]
    m = (lax.broadcasted_iota(jnp.int32, (_BR, G), 1) == b).astype(jnp.float32)
    gmax_n = jnp.dot(m, gmax_ref[...], precision=_HIGH)
    ex = jnp.where(b < G, jnp.exp(g_ref[...] - gmax_n), 0.0)
    dn = (((0,), (0,)), ((), ()))
    den_ref[...] += lax.dot_general(m, ex, dn, precision=_HIGH)
    u_ref[...] += lax.dot_general(m, ex * h, dn, precision=_HIGH)

    @pl.when(i == pl.num_programs(0) - 1)
    def _():
        pooled = u_ref[...] / (den_ref[...] + 1e-16)
        out_ref[...] = jnp.dot(
            pooled, pw_ref[...], precision=_HIGH) + pb_ref[...]


def _attention_pool(h, gate, batch_p, gmax_t, pred_w_p, pred_b_p):
    return pl.pallas_call(
        _pool_body,
        grid=(NPAD // _BR,),
        in_specs=[
            pl.BlockSpec((2, _BR, DH), lambda i: (0, i, 0)),
            pl.BlockSpec((_BR, 1), lambda i: (i, 0)),
            pl.BlockSpec((_BR, 1), lambda i: (i, 0)),
            pl.BlockSpec((G, 1), lambda i: (0, 0)),
            pl.BlockSpec((DP, DP), lambda i: (0, 0)),
            pl.BlockSpec((1, DP), lambda i: (0, 0)),
        ],
        out_specs=pl.BlockSpec((G, DP), lambda i: (0, 0)),
        out_shape=jax.ShapeDtypeStruct((G, DP), jnp.float32),
        scratch_shapes=[
            pltpu.VMEM((G, 1), jnp.float32),
            pltpu.VMEM((G, DP), jnp.float32),
        ],
        compiler_params=pltpu.CompilerParams(
            dimension_semantics=("arbitrary",)),
    )(h, gate, batch_p, gmax_t, pred_w_p, pred_b_p)


def kernel(x, edge_index, edge_attr, batch, atom_emb, chir_emb, bond_type_emb,
           bond_dir_emb, W1, b1, W2, b2, bn_gamma, bn_beta, gate_w, gate_b,
           pred_w, pred_b):
    eps = 1e-5
    i32 = jnp.int32
    f32 = jnp.float32

    x_p = jnp.zeros((NPAD, 2), i32).at[:N].set(x.astype(i32))
    srcp = jnp.zeros((EPAD,), i32).at[:E].set(edge_index[0].astype(i32))
    dstp = jnp.full((EPAD,), N, i32).at[:E].set(edge_index[1].astype(i32))
    ea0p = jnp.zeros((EPAD,), i32).at[:E].set(edge_attr[:, 0].astype(i32))
    ea1p = jnp.zeros((EPAD,), i32).at[:E].set(edge_attr[:, 1].astype(i32))
    batch_p = jnp.full((NPAD, 1), G, i32).at[:N, 0].set(batch.astype(i32))

    atom_emb_p = jnp.zeros((128, DP), f32).at[:120, :D].set(atom_emb)
    chir_emb_p = jnp.zeros((8, DP), f32).at[:3, :D].set(chir_emb)
    ctabs = (jnp.zeros((L, CNT, DP), f32)
             .at[:, :6, :D].set(bond_type_emb)
             .at[:, 6:9, :D].set(bond_dir_emb))
    sls = jnp.zeros((L, 1, DP), f32).at[:, 0, :D].set(
        bond_type_emb[:, 4, :] + bond_dir_emb[:, 0, :])
    W1p = jnp.zeros((L, DP, HID), f32).at[:, :D, :2 * D].set(W1)
    b1p = jnp.zeros((L, 1, HID), f32).at[:, 0, :2 * D].set(b1)
    W2p = jnp.zeros((L, HID, DP), f32).at[:, :2 * D, :D].set(W2)
    b2p = jnp.zeros((L, 1, DP), f32).at[:, 0, :D].set(b2)
    scales = jnp.zeros((L, 1, DP), f32).at[:, 0, :D].set(
        bn_gamma / jnp.sqrt(1.0 + eps))
    betas = jnp.zeros((L, 1, DP), f32).at[:, 0, :D].set(bn_beta)
    gate_w_p = jnp.zeros((DP, 128), f32).at[:D, 0:1].set(gate_w)
    pred_w_p = jnp.zeros((DP, DP), f32).at[:D, :D].set(pred_w)
    pred_b_p = jnp.zeros((1, DP), f32).at[0, :D].set(pred_b)

    zrows = jnp.zeros((16, DH), f32)
    zrows_c = jnp.zeros((64, CNT), f32)

    h = _init_nodes(x_p, atom_emb_p, chir_emb_p)
    C = _edge_counts(ea0p, ea1p, dstp, zrows_c)
    for l in range(L):
        P = _neighbor_sum(h[0], h[1], srcp, dstp, zrows)
        h = _gin_layer(P, h, C, ctabs[l], sls[l], W1p[l], b1p[l], W2p[l],
                       b2p[l], scales[l], betas[l], relu=(l < L - 1))
    gate, gmax = _gate_sweep(h, batch_p, gate_w_p)
    out = _attention_pool(h, gate, batch_p, gmax.reshape(G, 1), pred_w_p,
                          pred_b_p)
    return out[:, None, :D]


# speedup vs baseline: 5.1110x; 1.0707x over previous
import functools

import jax
import jax.numpy as jnp
from jax import lax
from jax.experimental import pallas as pl
from jax.experimental.pallas import tpu as pltpu
from jax.experimental.pallas import tpu_sc as plsc

N = 10000
NPAD = 10240
D = 300
DP = 320
DH = 160
E = 160000
EPAD = 163840
L = 5
G = 256
HID = 640
CNT = 16
NC, NS = 2, 16
CHUNK = 128
RPT = NPAD // NS

_HIGH = lax.Precision.HIGHEST


@functools.cache
def _sc_mesh():
    return plsc.VectorSubcoreMesh(
        core_axis_name="c", subcore_axis_name="s", num_cores=NC,
        num_subcores=NS)


CH2 = 64


def _neigh_body_fn(h, src, dst, zrows, out, acc, zbuf,
                   is0, id0, is1, id1, rows0, rows1,
                   gsem0, gsem1, ssem0, ssem1):
    cid = lax.axis_index("c")
    sid = lax.axis_index("s")

    pltpu.sync_copy(zrows, zbuf)

    def zbody(i, carry):
        pltpu.sync_copy(zbuf, acc.at[pl.ds(sid * RPT + i * 16, 16)])
        return carry

    lax.fori_loop(0, RPT // 16, zbody, 0)
    plsc.subcore_barrier()

    ept = EPAD // NS

    def obody(o, carry):
        b0 = sid * ept + (2 * o) * CH2
        b1 = b0 + CH2
        pltpu.sync_copy(src.at[pl.ds(b0, CH2)], is0)
        pltpu.sync_copy(dst.at[pl.ds(b0, CH2)], id0)
        g0 = pltpu.async_copy(h.at[cid].at[is0], rows0, gsem0)
        pltpu.sync_copy(src.at[pl.ds(b1, CH2)], is1)
        pltpu.sync_copy(dst.at[pl.ds(b1, CH2)], id1)
        g1 = pltpu.async_copy(h.at[cid].at[is1], rows1, gsem1)
        g0.wait()
        s0 = pltpu.async_copy(rows0, acc.at[id0], ssem0, add=True)
        g1.wait()
        s1 = pltpu.async_copy(rows1, acc.at[id1], ssem1, add=True)
        s0.wait()
        s1.wait()
        return carry

    lax.fori_loop(0, ept // (2 * CH2), obody, 0)
    plsc.subcore_barrier()

    def dbody(i, carry):
        r = sid * RPT + i * 16
        pltpu.sync_copy(acc.at[pl.ds(r, 16)], zbuf)
        pltpu.sync_copy(zbuf, out.at[cid].at[pl.ds(r, 16)])
        return carry

    lax.fori_loop(0, RPT // 16, dbody, 0)


@functools.cache
def _build_neigh_kernel():
    return pl.kernel(
        _neigh_body_fn,
        out_type=jax.ShapeDtypeStruct((2, NPAD, DH), jnp.float32),
        mesh=_sc_mesh(),
        scratch_types=[
            pltpu.VMEM_SHARED((NPAD, DH), jnp.float32),
            pltpu.VMEM((16, DH), jnp.float32),
            pltpu.VMEM((CH2,), jnp.int32),
            pltpu.VMEM((CH2,), jnp.int32),
            pltpu.VMEM((CH2,), jnp.int32),
            pltpu.VMEM((CH2,), jnp.int32),
            pltpu.VMEM((CH2, DH), jnp.float32),
            pltpu.VMEM((CH2, DH), jnp.float32),
            pltpu.SemaphoreType.DMA,
            pltpu.SemaphoreType.DMA,
            pltpu.SemaphoreType.DMA,
            pltpu.SemaphoreType.DMA,
        ],
        compiler_params=pltpu.CompilerParams(use_tc_tiling_on_sc=False, needs_layout_passes=False),
    )


def _neighbor_sum(h, srcp, dstp, zrows):
    return _build_neigh_kernel()(h, srcp, dstp, zrows)


def _counts_body_fn(ea0, ea1, dst, zrows, out,
                    acc, zbuf, rowsc, idx_t, idx_r, idx_d):
    cid = lax.axis_index("c")
    sid = lax.axis_index("s")
    wid = sid * NC + cid

    pltpu.sync_copy(zrows, zbuf)

    def zbody(i, carry):
        pltpu.sync_copy(zbuf, acc.at[pl.ds(sid * RPT + i * 64, 64)])
        return carry

    lax.fori_loop(0, RPT // 64, zbody, 0)
    plsc.subcore_barrier()

    epw = EPAD // (NC * NS)
    ones = jnp.ones((16,), jnp.float32)

    def ebody(c, carry):
        b = wid * epw + c * CHUNK
        pltpu.sync_copy(ea0.at[pl.ds(b, CHUNK)], idx_t)
        pltpu.sync_copy(ea1.at[pl.ds(b, CHUNK)], idx_r)
        pltpu.sync_copy(dst.at[pl.ds(b, CHUNK)], idx_d)
        zero16 = jnp.zeros((16,), jnp.float32)
        for i in range(CHUNK):
            rowsc[i, :] = zero16
        for g in range(CHUNK // 16):
            rid = lax.iota(jnp.int32, 16) + g * 16
            t16 = idx_t[pl.ds(g * 16, 16)]
            plsc.addupdate_scatter(rowsc, [rid, t16], ones)
            r16 = idx_r[pl.ds(g * 16, 16)]
            plsc.addupdate_scatter(rowsc, [rid, r16 + 6], ones)
        pltpu.sync_copy(rowsc, acc.at[idx_d], add=True)
        return carry

    lax.fori_loop(0, epw // CHUNK, ebody, 0)
    plsc.subcore_barrier()

    def dbody(i, carry):
        r = sid * RPT + i * 64
        pltpu.sync_copy(acc.at[pl.ds(r, 64)], zbuf)
        pltpu.sync_copy(zbuf, out.at[cid].at[pl.ds(r, 64)])
        return carry

    lax.fori_loop(0, RPT // 64, dbody, 0)


@functools.cache
def _build_counts_kernel():
    return pl.kernel(
        _counts_body_fn,
        out_type=jax.ShapeDtypeStruct((2, NPAD, CNT), jnp.float32),
        mesh=_sc_mesh(),
        scratch_types=[
            pltpu.VMEM_SHARED((NPAD, CNT), jnp.float32),
            pltpu.VMEM((64, CNT), jnp.float32),
            pltpu.VMEM((CHUNK, CNT), jnp.float32),
            pltpu.VMEM((CHUNK,), jnp.int32),
            pltpu.VMEM((CHUNK,), jnp.int32),
            pltpu.VMEM((CHUNK,), jnp.int32),
        ],
        compiler_params=pltpu.CompilerParams(use_tc_tiling_on_sc=False, needs_layout_passes=False),
    )


def _edge_counts(ea0p, ea1p, dstp, zrows):
    return _build_counts_kernel()(ea0p, ea1p, dstp, zrows)


_BR = 1024


def _init_body(x_ref, ae_ref, ce_ref, out_ref):
    xb = x_ref[...]
    a = xb[:, 0:1]
    c = xb[:, 1:2]
    oh_a = (lax.broadcasted_iota(jnp.int32, (_BR, 128), 1) == a).astype(jnp.float32)
    oh_c = (lax.broadcasted_iota(jnp.int32, (_BR, 8), 1) == c).astype(jnp.float32)
    h = jnp.dot(oh_a, ae_ref[...], precision=_HIGH) + jnp.dot(
        oh_c, ce_ref[...], precision=_HIGH)
    out_ref[0, :, :] = h[:, :DH]
    out_ref[1, :, :] = h[:, DH:]


def _init_nodes(x_p, atom_emb_p, chir_emb_p):
    return pl.pallas_call(
        _init_body,
        grid=(NPAD // _BR,),
        in_specs=[
            pl.BlockSpec((_BR, 2), lambda i: (i, 0)),
            pl.BlockSpec((128, DP), lambda i: (0, 0)),
            pl.BlockSpec((8, DP), lambda i: (0, 0)),
        ],
        out_specs=pl.BlockSpec((2, _BR, DH), lambda i: (0, i, 0)),
        out_shape=jax.ShapeDtypeStruct((2, NPAD, DH), jnp.float32),
        compiler_params=pltpu.CompilerParams(
            dimension_semantics=("parallel",)),
    )(x_p, atom_emb_p, chir_emb_p)


def _layer_body(relu, p_ref, h_ref, c_ref, ctab_ref, sl_ref, w1_ref, b1_ref,
                w2_ref, b2_ref, sc_ref, be_ref, out_ref):
    aggA = p_ref[0] + h_ref[0]
    aggB = p_ref[1] + h_ref[1]
    agg = jnp.concatenate([aggA, aggB], axis=1)
    cnt = c_ref[0] + c_ref[1]
    agg = agg + jnp.dot(cnt, ctab_ref[...], precision=_HIGH) + sl_ref[...]
    z = jnp.maximum(jnp.dot(agg, w1_ref[...], precision=_HIGH) + b1_ref[...], 0.0)
    z = jnp.dot(z, w2_ref[...], precision=_HIGH) + b2_ref[...]
    z = z * sc_ref[...] + be_ref[...]
    if relu:
        z = jnp.maximum(z, 0.0)
    out_ref[0, :, :] = z[:, :DH]
    out_ref[1, :, :] = z[:, DH:]


def _gin_layer(P, h, C, ctab, sl, w1, b1, w2, b2, scale, beta, relu):
    return pl.pallas_call(
        functools.partial(_layer_body, relu),
        grid=(NPAD // _BR,),
        in_specs=[
            pl.BlockSpec((2, _BR, DH), lambda i: (0, i, 0)),
            pl.BlockSpec((2, _BR, DH), lambda i: (0, i, 0)),
            pl.BlockSpec((2, _BR, CNT), lambda i: (0, i, 0)),
            pl.BlockSpec((CNT, DP), lambda i: (0, 0)),
            pl.BlockSpec((1, DP), lambda i: (0, 0)),
            pl.BlockSpec((DP, HID), lambda i: (0, 0)),
            pl.BlockSpec((1, HID), lambda i: (0, 0)),
            pl.BlockSpec((HID, DP), lambda i: (0, 0)),
            pl.BlockSpec((1, DP), lambda i: (0, 0)),
            pl.BlockSpec((1, DP), lambda i: (0, 0)),
            pl.BlockSpec((1, DP), lambda i: (0, 0)),
        ],
        out_specs=pl.BlockSpec((2, _BR, DH), lambda i: (0, i, 0)),
        out_shape=jax.ShapeDtypeStruct((2, NPAD, DH), jnp.float32),
        compiler_params=pltpu.CompilerParams(
            dimension_semantics=("parallel",)),
    )(P, h, C, ctab, sl, w1, b1, w2, b2, scale, beta)


def _gate_body(h_ref, b_ref, gw_ref, gate_ref, gmax_ref, acc_ref):
    i = pl.program_id(0)
    h = jnp.concatenate([h_ref[0], h_ref[1]], axis=1)
    gate = jnp.dot(h, gw_ref[...], precision=_HIGH)[:, 0:1]
    gate_ref[...] = gate
    seg = lax.broadcasted_iota(jnp.int32, (_BR, G), 1) == b_ref[...]
    bmax = jnp.max(jnp.where(seg, gate, -jnp.inf), axis=0, keepdims=True)

    @pl.when(i == 0)
    def _():
        acc_ref[...] = jnp.full((8, G), -jnp.inf, jnp.float32)

    acc_ref[0:1, :] = jnp.maximum(acc_ref[0:1, :], bmax)
    gmax_ref[...] = jnp.where(
        jnp.isfinite(acc_ref[0:1, :]), acc_ref[0:1, :], 0.0)


def _gate_sweep(h, batch_p, gate_w_p):
    return pl.pallas_call(
        _gate_body,
        grid=(NPAD // _BR,),
        in_specs=[
            pl.BlockSpec((2, _BR, DH), lambda i: (0, i, 0)),
            pl.BlockSpec((_BR, 1), lambda i: (i, 0)),
            pl.BlockSpec((DP, 128), lambda i: (0, 0)),
        ],
        out_specs=[
            pl.BlockSpec((_BR, 1), lambda i: (i, 0)),
            pl.BlockSpec((1, G), lambda i: (0, 0)),
        ],
        out_shape=[
            jax.ShapeDtypeStruct((NPAD, 1), jnp.float32),
            jax.ShapeDtypeStruct((1, G), jnp.float32),
        ],
        scratch_shapes=[pltpu.VMEM((8, G), jnp.float32)],
        compiler_params=pltpu.CompilerParams(
            dimension_semantics=("arbitrary",)),
    )(h, batch_p, gate_w_p)


def _pool_body(h_ref, g_ref, b_ref, gmax_ref, pw_ref, pb_ref, out_ref,
               den_ref, u_ref):
    i = pl.program_id(0)

    @pl.when(i == 0)
    def _():
        den_ref[...] = jnp.zeros_like(den_ref)
        u_ref[...] = jnp.zeros_like(u_ref)

    h = jnp.concatenate([h_ref[0], h_ref[1]], axis=1)
    b = b_ref[...]
    m = (lax.broadcasted_iota(jnp.int32, (_BR, G), 1) == b).astype(jnp.float32)
    gmax_n = jnp.dot(m, gmax_ref[...], precision=_HIGH)
    ex = jnp.where(b < G, jnp.exp(g_ref[...] - gmax_n), 0.0)
    dn = (((0,), (0,)), ((), ()))
    den_ref[...] += lax.dot_general(m, ex, dn, precision=_HIGH)
    u_ref[...] += lax.dot_general(m, ex * h, dn, precision=_HIGH)

    @pl.when(i == pl.num_programs(0) - 1)
    def _():
        pooled = u_ref[...] / (den_ref[...] + 1e-16)
        out_ref[...] = jnp.dot(
            pooled, pw_ref[...], precision=_HIGH) + pb_ref[...]


def _attention_pool(h, gate, batch_p, gmax_t, pred_w_p, pred_b_p):
    return pl.pallas_call(
        _pool_body,
        grid=(NPAD // _BR,),
        in_specs=[
            pl.BlockSpec((2, _BR, DH), lambda i: (0, i, 0)),
            pl.BlockSpec((_BR, 1), lambda i: (i, 0)),
            pl.BlockSpec((_BR, 1), lambda i: (i, 0)),
            pl.BlockSpec((G, 1), lambda i: (0, 0)),
            pl.BlockSpec((DP, DP), lambda i: (0, 0)),
            pl.BlockSpec((1, DP), lambda i: (0, 0)),
        ],
        out_specs=pl.BlockSpec((G, DP), lambda i: (0, 0)),
        out_shape=jax.ShapeDtypeStruct((G, DP), jnp.float32),
        scratch_shapes=[
            pltpu.VMEM((G, 1), jnp.float32),
            pltpu.VMEM((G, DP), jnp.float32),
        ],
        compiler_params=pltpu.CompilerParams(
            dimension_semantics=("arbitrary",)),
    )(h, gate, batch_p, gmax_t, pred_w_p, pred_b_p)


def kernel(x, edge_index, edge_attr, batch, atom_emb, chir_emb, bond_type_emb,
           bond_dir_emb, W1, b1, W2, b2, bn_gamma, bn_beta, gate_w, gate_b,
           pred_w, pred_b):
    eps = 1e-5
    i32 = jnp.int32
    f32 = jnp.float32

    x_p = jnp.zeros((NPAD, 2), i32).at[:N].set(x.astype(i32))
    srcp = jnp.zeros((EPAD,), i32).at[:E].set(edge_index[0].astype(i32))
    dstp = jnp.full((EPAD,), N, i32).at[:E].set(edge_index[1].astype(i32))
    ea0p = jnp.zeros((EPAD,), i32).at[:E].set(edge_attr[:, 0].astype(i32))
    ea1p = jnp.zeros((EPAD,), i32).at[:E].set(edge_attr[:, 1].astype(i32))
    batch_p = jnp.full((NPAD, 1), G, i32).at[:N, 0].set(batch.astype(i32))

    atom_emb_p = jnp.zeros((128, DP), f32).at[:120, :D].set(atom_emb)
    chir_emb_p = jnp.zeros((8, DP), f32).at[:3, :D].set(chir_emb)
    ctabs = (jnp.zeros((L, CNT, DP), f32)
             .at[:, :6, :D].set(bond_type_emb)
             .at[:, 6:9, :D].set(bond_dir_emb))
    sls = jnp.zeros((L, 1, DP), f32).at[:, 0, :D].set(
        bond_type_emb[:, 4, :] + bond_dir_emb[:, 0, :])
    W1p = jnp.zeros((L, DP, HID), f32).at[:, :D, :2 * D].set(W1)
    b1p = jnp.zeros((L, 1, HID), f32).at[:, 0, :2 * D].set(b1)
    W2p = jnp.zeros((L, HID, DP), f32).at[:, :2 * D, :D].set(W2)
    b2p = jnp.zeros((L, 1, DP), f32).at[:, 0, :D].set(b2)
    scales = jnp.zeros((L, 1, DP), f32).at[:, 0, :D].set(
        bn_gamma / jnp.sqrt(1.0 + eps))
    betas = jnp.zeros((L, 1, DP), f32).at[:, 0, :D].set(bn_beta)
    gate_w_p = jnp.zeros((DP, 128), f32).at[:D, 0:1].set(gate_w)
    pred_w_p = jnp.zeros((DP, DP), f32).at[:D, :D].set(pred_w)
    pred_b_p = jnp.zeros((1, DP), f32).at[0, :D].set(pred_b)

    zrows = jnp.zeros((16, DH), f32)
    zrows_c = jnp.zeros((64, CNT), f32)

    h = _init_nodes(x_p, atom_emb_p, chir_emb_p)
    C = _edge_counts(ea0p, ea1p, dstp, zrows_c)
    for l in range(L):
        P = _neighbor_sum(h, srcp, dstp, zrows)
        h = _gin_layer(P, h, C, ctabs[l], sls[l], W1p[l], b1p[l], W2p[l],
                       b2p[l], scales[l], betas[l], relu=(l < L - 1))
    gate, gmax = _gate_sweep(h, batch_p, gate_w_p)
    out = _attention_pool(h, gate, batch_p, gmax.reshape(G, 1), pred_w_p,
                          pred_b_p)
    return out[:, None, :D]


# speedup vs baseline: 5.6421x; 1.1039x over previous
import functools

import jax
import jax.numpy as jnp
from jax import lax
from jax.experimental import pallas as pl
from jax.experimental.pallas import tpu as pltpu
from jax.experimental.pallas import tpu_sc as plsc

N = 10000
NPAD = 10240
D = 300
DP = 320
DH = 160
E = 160000
EPAD = 163840
L = 5
G = 256
HID = 640
CNT = 16
NC, NS = 2, 16
CHUNK = 128
RPT = NPAD // NS

_HIGH = lax.Precision.HIGHEST


@functools.cache
def _sc_mesh():
    return plsc.VectorSubcoreMesh(
        core_axis_name="c", subcore_axis_name="s", num_cores=NC,
        num_subcores=NS)


CH2 = 64


NCHB = 16


def _neigh_body_fn(h, src2, dst2, zrows, out, acc, zbuf,
                   isb, idb, rows0, rows1,
                   gsem0, gsem1, ssem0, ssem1):
    cid = lax.axis_index("c")
    sid = lax.axis_index("s")

    pltpu.sync_copy(zrows, zbuf)

    def zbody(i, carry):
        pltpu.sync_copy(zbuf, acc.at[pl.ds(sid * RPT + i * 16, 16)])
        return carry

    lax.fori_loop(0, RPT // 16, zbody, 0)
    plsc.subcore_barrier()

    ept = EPAD // NS
    nblk = ept // (NCHB * CH2)
    rows_ = (rows0, rows1)
    gsem_ = (gsem0, gsem1)
    ssem_ = (ssem0, ssem1)

    def obody(o, carry):
        blk = sid * nblk + o
        pltpu.sync_copy(src2.at[blk], isb)
        pltpu.sync_copy(dst2.at[blk], idb)
        for j in range(NCHB):
            sl = j % 2

            def drain():
                pltpu.make_async_copy(
                    rows_[sl], acc.at[idb.at[j]], ssem_[sl]).wait()

            if j >= 2:
                drain()
            else:
                pl.when(o > 0)(drain)
            pltpu.async_copy(
                h.at[cid].at[isb.at[j]], rows_[sl], gsem_[sl]).wait()
            pltpu.async_copy(
                rows_[sl], acc.at[idb.at[j]], ssem_[sl], add=True)
        return carry

    lax.fori_loop(0, nblk, obody, 0)
    pltpu.make_async_copy(rows0, acc.at[idb.at[0]], ssem0).wait()
    pltpu.make_async_copy(rows1, acc.at[idb.at[1]], ssem1).wait()
    plsc.subcore_barrier()

    def dbody(i, carry):
        r = sid * RPT + i * 16
        pltpu.sync_copy(acc.at[pl.ds(r, 16)], zbuf)
        pltpu.sync_copy(zbuf, out.at[cid].at[pl.ds(r, 16)])
        return carry

    lax.fori_loop(0, RPT // 16, dbody, 0)


@functools.cache
def _build_neigh_kernel():
    return pl.kernel(
        _neigh_body_fn,
        out_type=jax.ShapeDtypeStruct((2, NPAD, DH), jnp.float32),
        mesh=_sc_mesh(),
        scratch_types=[
            pltpu.VMEM_SHARED((NPAD, DH), jnp.float32),
            pltpu.VMEM((16, DH), jnp.float32),
            pltpu.VMEM((NCHB, CH2), jnp.int32),
            pltpu.VMEM((NCHB, CH2), jnp.int32),
            pltpu.VMEM((CH2, DH), jnp.float32),
            pltpu.VMEM((CH2, DH), jnp.float32),
            pltpu.SemaphoreType.DMA,
            pltpu.SemaphoreType.DMA,
            pltpu.SemaphoreType.DMA,
            pltpu.SemaphoreType.DMA,
        ],
        compiler_params=pltpu.CompilerParams(use_tc_tiling_on_sc=False, needs_layout_passes=False),
    )


def _neighbor_sum(h, srcp, dstp, zrows):
    src2 = srcp.reshape(-1, NCHB, CH2)
    dst2 = dstp.reshape(-1, NCHB, CH2)
    return _build_neigh_kernel()(h, src2, dst2, zrows)


def _counts_body_fn(ea0, ea1, dst, zrows, out,
                    acc, zbuf, rowsc, idx_t, idx_r, idx_d):
    cid = lax.axis_index("c")
    sid = lax.axis_index("s")
    wid = sid * NC + cid

    pltpu.sync_copy(zrows, zbuf)

    def zbody(i, carry):
        pltpu.sync_copy(zbuf, acc.at[pl.ds(sid * RPT + i * 64, 64)])
        return carry

    lax.fori_loop(0, RPT // 64, zbody, 0)
    plsc.subcore_barrier()

    epw = EPAD // (NC * NS)
    ones = jnp.ones((16,), jnp.float32)

    def ebody(c, carry):
        b = wid * epw + c * CHUNK
        pltpu.sync_copy(ea0.at[pl.ds(b, CHUNK)], idx_t)
        pltpu.sync_copy(ea1.at[pl.ds(b, CHUNK)], idx_r)
        pltpu.sync_copy(dst.at[pl.ds(b, CHUNK)], idx_d)
        zero16 = jnp.zeros((16,), jnp.float32)
        for i in range(CHUNK):
            rowsc[i, :] = zero16
        for g in range(CHUNK // 16):
            rid = lax.iota(jnp.int32, 16) + g * 16
            t16 = idx_t[pl.ds(g * 16, 16)]
            plsc.addupdate_scatter(rowsc, [rid, t16], ones)
            r16 = idx_r[pl.ds(g * 16, 16)]
            plsc.addupdate_scatter(rowsc, [rid, r16 + 6], ones)
        pltpu.sync_copy(rowsc, acc.at[idx_d], add=True)
        return carry

    lax.fori_loop(0, epw // CHUNK, ebody, 0)
    plsc.subcore_barrier()

    def dbody(i, carry):
        r = sid * RPT + i * 64
        pltpu.sync_copy(acc.at[pl.ds(r, 64)], zbuf)
        pltpu.sync_copy(zbuf, out.at[cid].at[pl.ds(r, 64)])
        return carry

    lax.fori_loop(0, RPT // 64, dbody, 0)


@functools.cache
def _build_counts_kernel():
    return pl.kernel(
        _counts_body_fn,
        out_type=jax.ShapeDtypeStruct((2, NPAD, CNT), jnp.float32),
        mesh=_sc_mesh(),
        scratch_types=[
            pltpu.VMEM_SHARED((NPAD, CNT), jnp.float32),
            pltpu.VMEM((64, CNT), jnp.float32),
            pltpu.VMEM((CHUNK, CNT), jnp.float32),
            pltpu.VMEM((CHUNK,), jnp.int32),
            pltpu.VMEM((CHUNK,), jnp.int32),
            pltpu.VMEM((CHUNK,), jnp.int32),
        ],
        compiler_params=pltpu.CompilerParams(use_tc_tiling_on_sc=False, needs_layout_passes=False),
    )


def _edge_counts(ea0p, ea1p, dstp, zrows):
    return _build_counts_kernel()(ea0p, ea1p, dstp, zrows)


_BR = 1024


def _init_body(x_ref, ae_ref, ce_ref, out_ref):
    xb = x_ref[...]
    a = xb[:, 0:1]
    c = xb[:, 1:2]
    oh_a = (lax.broadcasted_iota(jnp.int32, (_BR, 128), 1) == a).astype(jnp.float32)
    oh_c = (lax.broadcasted_iota(jnp.int32, (_BR, 8), 1) == c).astype(jnp.float32)
    h = jnp.dot(oh_a, ae_ref[...], precision=_HIGH) + jnp.dot(
        oh_c, ce_ref[...], precision=_HIGH)
    out_ref[0, :, :] = h[:, :DH]
    out_ref[1, :, :] = h[:, DH:]


def _init_nodes(x_p, atom_emb_p, chir_emb_p):
    return pl.pallas_call(
        _init_body,
        grid=(NPAD // _BR,),
        in_specs=[
            pl.BlockSpec((_BR, 2), lambda i: (i, 0)),
            pl.BlockSpec((128, DP), lambda i: (0, 0)),
            pl.BlockSpec((8, DP), lambda i: (0, 0)),
        ],
        out_specs=pl.BlockSpec((2, _BR, DH), lambda i: (0, i, 0)),
        out_shape=jax.ShapeDtypeStruct((2, NPAD, DH), jnp.float32),
        compiler_params=pltpu.CompilerParams(
            dimension_semantics=("parallel",)),
    )(x_p, atom_emb_p, chir_emb_p)


def _layer_body(relu, p_ref, h_ref, c_ref, ctab_ref, sl_ref, w1_ref, b1_ref,
                w2_ref, b2_ref, sc_ref, be_ref, out_ref):
    aggA = p_ref[0] + h_ref[0]
    aggB = p_ref[1] + h_ref[1]
    agg = jnp.concatenate([aggA, aggB], axis=1)
    cnt = c_ref[0] + c_ref[1]
    agg = agg + jnp.dot(cnt, ctab_ref[...], precision=_HIGH) + sl_ref[...]
    z = jnp.maximum(jnp.dot(agg, w1_ref[...], precision=_HIGH) + b1_ref[...], 0.0)
    z = jnp.dot(z, w2_ref[...], precision=_HIGH) + b2_ref[...]
    z = z * sc_ref[...] + be_ref[...]
    if relu:
        z = jnp.maximum(z, 0.0)
    out_ref[0, :, :] = z[:, :DH]
    out_ref[1, :, :] = z[:, DH:]


def _gin_layer(P, h, C, ctab, sl, w1, b1, w2, b2, scale, beta, relu):
    return pl.pallas_call(
        functools.partial(_layer_body, relu),
        grid=(NPAD // _BR,),
        in_specs=[
            pl.BlockSpec((2, _BR, DH), lambda i: (0, i, 0)),
            pl.BlockSpec((2, _BR, DH), lambda i: (0, i, 0)),
            pl.BlockSpec((2, _BR, CNT), lambda i: (0, i, 0)),
            pl.BlockSpec((CNT, DP), lambda i: (0, 0)),
            pl.BlockSpec((1, DP), lambda i: (0, 0)),
            pl.BlockSpec((DP, HID), lambda i: (0, 0)),
            pl.BlockSpec((1, HID), lambda i: (0, 0)),
            pl.BlockSpec((HID, DP), lambda i: (0, 0)),
            pl.BlockSpec((1, DP), lambda i: (0, 0)),
            pl.BlockSpec((1, DP), lambda i: (0, 0)),
            pl.BlockSpec((1, DP), lambda i: (0, 0)),
        ],
        out_specs=pl.BlockSpec((2, _BR, DH), lambda i: (0, i, 0)),
        out_shape=jax.ShapeDtypeStruct((2, NPAD, DH), jnp.float32),
        compiler_params=pltpu.CompilerParams(
            dimension_semantics=("parallel",)),
    )(P, h, C, ctab, sl, w1, b1, w2, b2, scale, beta)


def _gate_body(h_ref, b_ref, gw_ref, gate_ref, gmax_ref, acc_ref):
    i = pl.program_id(0)
    h = jnp.concatenate([h_ref[0], h_ref[1]], axis=1)
    gate = jnp.dot(h, gw_ref[...], precision=_HIGH)[:, 0:1]
    gate_ref[...] = gate
    seg = lax.broadcasted_iota(jnp.int32, (_BR, G), 1) == b_ref[...]
    bmax = jnp.max(jnp.where(seg, gate, -jnp.inf), axis=0, keepdims=True)

    @pl.when(i == 0)
    def _():
        acc_ref[...] = jnp.full((8, G), -jnp.inf, jnp.float32)

    acc_ref[0:1, :] = jnp.maximum(acc_ref[0:1, :], bmax)
    gmax_ref[...] = jnp.where(
        jnp.isfinite(acc_ref[0:1, :]), acc_ref[0:1, :], 0.0)


def _gate_sweep(h, batch_p, gate_w_p):
    return pl.pallas_call(
        _gate_body,
        grid=(NPAD // _BR,),
        in_specs=[
            pl.BlockSpec((2, _BR, DH), lambda i: (0, i, 0)),
            pl.BlockSpec((_BR, 1), lambda i: (i, 0)),
            pl.BlockSpec((DP, 128), lambda i: (0, 0)),
        ],
        out_specs=[
            pl.BlockSpec((_BR, 1), lambda i: (i, 0)),
            pl.BlockSpec((1, G), lambda i: (0, 0)),
        ],
        out_shape=[
            jax.ShapeDtypeStruct((NPAD, 1), jnp.float32),
            jax.ShapeDtypeStruct((1, G), jnp.float32),
        ],
        scratch_shapes=[pltpu.VMEM((8, G), jnp.float32)],
        compiler_params=pltpu.CompilerParams(
            dimension_semantics=("arbitrary",)),
    )(h, batch_p, gate_w_p)


def _pool_body(h_ref, g_ref, b_ref, gmax_ref, pw_ref, pb_ref, out_ref,
               den_ref, u_ref):
    i = pl.program_id(0)

    @pl.when(i == 0)
    def _():
        den_ref[...] = jnp.zeros_like(den_ref)
        u_ref[...] = jnp.zeros_like(u_ref)

    h = jnp.concatenate([h_ref[0], h_ref[1]], axis=1)
    b = b_ref[...]
    m = (lax.broadcasted_iota(jnp.int32, (_BR, G), 1) == b).astype(jnp.float32)
    gmax_n = jnp.dot(m, gmax_ref[...], precision=_HIGH)
    ex = jnp.where(b < G, jnp.exp(g_ref[...] - gmax_n), 0.0)
    dn = (((0,), (0,)), ((), ()))
    den_ref[...] += lax.dot_general(m, ex, dn, precision=_HIGH)
    u_ref[...] += lax.dot_general(m, ex * h, dn, precision=_HIGH)

    @pl.when(i == pl.num_programs(0) - 1)
    def _():
        pooled = u_ref[...] / (den_ref[...] + 1e-16)
        out_ref[...] = jnp.dot(
            pooled, pw_ref[...], precision=_HIGH) + pb_ref[...]


def _attention_pool(h, gate, batch_p, gmax_t, pred_w_p, pred_b_p):
    return pl.pallas_call(
        _pool_body,
        grid=(NPAD // _BR,),
        in_specs=[
            pl.BlockSpec((2, _BR, DH), lambda i: (0, i, 0)),
            pl.BlockSpec((_BR, 1), lambda i: (i, 0)),
            pl.BlockSpec((_BR, 1), lambda i: (i, 0)),
            pl.BlockSpec((G, 1), lambda i: (0, 0)),
            pl.BlockSpec((DP, DP), lambda i: (0, 0)),
            pl.BlockSpec((1, DP), lambda i: (0, 0)),
        ],
        out_specs=pl.BlockSpec((G, DP), lambda i: (0, 0)),
        out_shape=jax.ShapeDtypeStruct((G, DP), jnp.float32),
        scratch_shapes=[
            pltpu.VMEM((G, 1), jnp.float32),
            pltpu.VMEM((G, DP), jnp.float32),
        ],
        compiler_params=pltpu.CompilerParams(
            dimension_semantics=("arbitrary",)),
    )(h, gate, batch_p, gmax_t, pred_w_p, pred_b_p)


def kernel(x, edge_index, edge_attr, batch, atom_emb, chir_emb, bond_type_emb,
           bond_dir_emb, W1, b1, W2, b2, bn_gamma, bn_beta, gate_w, gate_b,
           pred_w, pred_b):
    eps = 1e-5
    i32 = jnp.int32
    f32 = jnp.float32

    x_p = jnp.zeros((NPAD, 2), i32).at[:N].set(x.astype(i32))
    srcp = jnp.zeros((EPAD,), i32).at[:E].set(edge_index[0].astype(i32))
    dstp = jnp.full((EPAD,), N, i32).at[:E].set(edge_index[1].astype(i32))
    ea0p = jnp.zeros((EPAD,), i32).at[:E].set(edge_attr[:, 0].astype(i32))
    ea1p = jnp.zeros((EPAD,), i32).at[:E].set(edge_attr[:, 1].astype(i32))
    batch_p = jnp.full((NPAD, 1), G, i32).at[:N, 0].set(batch.astype(i32))

    atom_emb_p = jnp.zeros((128, DP), f32).at[:120, :D].set(atom_emb)
    chir_emb_p = jnp.zeros((8, DP), f32).at[:3, :D].set(chir_emb)
    ctabs = (jnp.zeros((L, CNT, DP), f32)
             .at[:, :6, :D].set(bond_type_emb)
             .at[:, 6:9, :D].set(bond_dir_emb))
    sls = jnp.zeros((L, 1, DP), f32).at[:, 0, :D].set(
        bond_type_emb[:, 4, :] + bond_dir_emb[:, 0, :])
    W1p = jnp.zeros((L, DP, HID), f32).at[:, :D, :2 * D].set(W1)
    b1p = jnp.zeros((L, 1, HID), f32).at[:, 0, :2 * D].set(b1)
    W2p = jnp.zeros((L, HID, DP), f32).at[:, :2 * D, :D].set(W2)
    b2p = jnp.zeros((L, 1, DP), f32).at[:, 0, :D].set(b2)
    scales = jnp.zeros((L, 1, DP), f32).at[:, 0, :D].set(
        bn_gamma / jnp.sqrt(1.0 + eps))
    betas = jnp.zeros((L, 1, DP), f32).at[:, 0, :D].set(bn_beta)
    gate_w_p = jnp.zeros((DP, 128), f32).at[:D, 0:1].set(gate_w)
    pred_w_p = jnp.zeros((DP, DP), f32).at[:D, :D].set(pred_w)
    pred_b_p = jnp.zeros((1, DP), f32).at[0, :D].set(pred_b)

    zrows = jnp.zeros((16, DH), f32)
    zrows_c = jnp.zeros((64, CNT), f32)

    h = _init_nodes(x_p, atom_emb_p, chir_emb_p)
    C = _edge_counts(ea0p, ea1p, dstp, zrows_c)
    for l in range(L):
        P = _neighbor_sum(h, srcp, dstp, zrows)
        h = _gin_layer(P, h, C, ctabs[l], sls[l], W1p[l], b1p[l], W2p[l],
                       b2p[l], scales[l], betas[l], relu=(l < L - 1))
    gate, gmax = _gate_sweep(h, batch_p, gate_w_p)
    out = _attention_pool(h, gate, batch_p, gmax.reshape(G, 1), pred_w_p,
                          pred_b_p)
    return out[:, None, :D]


# speedup vs baseline: 5.7207x; 1.0139x over previous
import functools

import jax
import jax.numpy as jnp
from jax import lax
from jax.experimental import pallas as pl
from jax.experimental.pallas import tpu as pltpu
from jax.experimental.pallas import tpu_sc as plsc

N = 10000
NPAD = 10240
D = 300
DP = 320
DH = 160
E = 160000
EPAD = 163840
L = 5
G = 256
HID = 640
CNT = 16
NC, NS = 2, 16
CHUNK = 128
RPT = NPAD // NS

_HIGH = lax.Precision.HIGHEST


@functools.cache
def _sc_mesh():
    return plsc.VectorSubcoreMesh(
        core_axis_name="c", subcore_axis_name="s", num_cores=NC,
        num_subcores=NS)


CH2 = 64


NCHB = 16


def _neigh_body_fn(h, src2, dst2, zrows, out, acc,
                   isb, idb, rows0, rows1,
                   gsem0, gsem1, ssem0, ssem1):
    cid = lax.axis_index("c")
    sid = lax.axis_index("s")

    pltpu.sync_copy(zrows, rows0)
    for i in range(RPT // 64):
        pltpu.async_copy(
            rows0, acc.at[pl.ds(sid * RPT + i * 64, 64)], ssem0)
    for i in range(RPT // 64):
        pltpu.make_async_copy(
            rows0, acc.at[pl.ds(sid * RPT + i * 64, 64)], ssem0).wait()
    plsc.subcore_barrier()

    ept = EPAD // NS
    nblk = ept // (NCHB * CH2)
    rows_ = (rows0, rows1)
    gsem_ = (gsem0, gsem1)
    ssem_ = (ssem0, ssem1)

    def obody(o, carry):
        blk = sid * nblk + o
        pltpu.sync_copy(src2.at[blk], isb)
        pltpu.sync_copy(dst2.at[blk], idb)
        for j in range(NCHB):
            sl = j % 2

            def drain():
                pltpu.make_async_copy(
                    rows_[sl], acc.at[idb.at[j]], ssem_[sl]).wait()

            if j >= 2:
                drain()
            else:
                pl.when(o > 0)(drain)
            pltpu.async_copy(
                h.at[cid].at[isb.at[j]], rows_[sl], gsem_[sl]).wait()
            pltpu.async_copy(
                rows_[sl], acc.at[idb.at[j]], ssem_[sl], add=True)
        return carry

    lax.fori_loop(0, nblk, obody, 0)
    pltpu.make_async_copy(rows0, acc.at[idb.at[0]], ssem0).wait()
    pltpu.make_async_copy(rows1, acc.at[idb.at[1]], ssem1).wait()
    plsc.subcore_barrier()

    nds = RPT // 64
    for i in range(nds):
        buf = rows_[i % 2]
        r = sid * RPT + i * 64
        if i >= 2:
            pltpu.make_async_copy(
                buf, out.at[cid].at[pl.ds(r, 64)], ssem_[i % 2]).wait()
        pltpu.async_copy(acc.at[pl.ds(r, 64)], buf, gsem_[i % 2]).wait()
        pltpu.async_copy(buf, out.at[cid].at[pl.ds(r, 64)], ssem_[i % 2])
    for i in range(nds - 2, nds):
        r = sid * RPT + i * 64
        pltpu.make_async_copy(
            rows_[i % 2], out.at[cid].at[pl.ds(r, 64)], ssem_[i % 2]).wait()


@functools.cache
def _build_neigh_kernel():
    return pl.kernel(
        _neigh_body_fn,
        out_type=jax.ShapeDtypeStruct((2, NPAD, DH), jnp.float32),
        mesh=_sc_mesh(),
        scratch_types=[
            pltpu.VMEM_SHARED((NPAD, DH), jnp.float32),
            pltpu.VMEM((NCHB, CH2), jnp.int32),
            pltpu.VMEM((NCHB, CH2), jnp.int32),
            pltpu.VMEM((CH2, DH), jnp.float32),
            pltpu.VMEM((CH2, DH), jnp.float32),
            pltpu.SemaphoreType.DMA,
            pltpu.SemaphoreType.DMA,
            pltpu.SemaphoreType.DMA,
            pltpu.SemaphoreType.DMA,
        ],
        compiler_params=pltpu.CompilerParams(use_tc_tiling_on_sc=False, needs_layout_passes=False),
    )


def _neighbor_sum(h, srcp, dstp, zrows):
    src2 = srcp.reshape(-1, NCHB, CH2)
    dst2 = dstp.reshape(-1, NCHB, CH2)
    return _build_neigh_kernel()(h, src2, dst2, zrows)


def _counts_body_fn(ea0, ea1, dst, zrows, out,
                    acc, zbuf, rowsc, idx_t, idx_r, idx_d):
    cid = lax.axis_index("c")
    sid = lax.axis_index("s")
    wid = sid * NC + cid

    pltpu.sync_copy(zrows, zbuf)

    def zbody(i, carry):
        pltpu.sync_copy(zbuf, acc.at[pl.ds(sid * RPT + i * 64, 64)])
        return carry

    lax.fori_loop(0, RPT // 64, zbody, 0)
    plsc.subcore_barrier()

    epw = EPAD // (NC * NS)
    ones = jnp.ones((16,), jnp.float32)

    def ebody(c, carry):
        b = wid * epw + c * CHUNK
        pltpu.sync_copy(ea0.at[pl.ds(b, CHUNK)], idx_t)
        pltpu.sync_copy(ea1.at[pl.ds(b, CHUNK)], idx_r)
        pltpu.sync_copy(dst.at[pl.ds(b, CHUNK)], idx_d)
        zero16 = jnp.zeros((16,), jnp.float32)
        for i in range(CHUNK):
            rowsc[i, :] = zero16
        for g in range(CHUNK // 16):
            rid = lax.iota(jnp.int32, 16) + g * 16
            t16 = idx_t[pl.ds(g * 16, 16)]
            plsc.addupdate_scatter(rowsc, [rid, t16], ones)
            r16 = idx_r[pl.ds(g * 16, 16)]
            plsc.addupdate_scatter(rowsc, [rid, r16 + 6], ones)
        pltpu.sync_copy(rowsc, acc.at[idx_d], add=True)
        return carry

    lax.fori_loop(0, epw // CHUNK, ebody, 0)
    plsc.subcore_barrier()

    def dbody(i, carry):
        r = sid * RPT + i * 64
        pltpu.sync_copy(acc.at[pl.ds(r, 64)], zbuf)
        pltpu.sync_copy(zbuf, out.at[cid].at[pl.ds(r, 64)])
        return carry

    lax.fori_loop(0, RPT // 64, dbody, 0)


@functools.cache
def _build_counts_kernel():
    return pl.kernel(
        _counts_body_fn,
        out_type=jax.ShapeDtypeStruct((2, NPAD, CNT), jnp.float32),
        mesh=_sc_mesh(),
        scratch_types=[
            pltpu.VMEM_SHARED((NPAD, CNT), jnp.float32),
            pltpu.VMEM((64, CNT), jnp.float32),
            pltpu.VMEM((CHUNK, CNT), jnp.float32),
            pltpu.VMEM((CHUNK,), jnp.int32),
            pltpu.VMEM((CHUNK,), jnp.int32),
            pltpu.VMEM((CHUNK,), jnp.int32),
        ],
        compiler_params=pltpu.CompilerParams(use_tc_tiling_on_sc=False, needs_layout_passes=False),
    )


def _edge_counts(ea0p, ea1p, dstp, zrows):
    return _build_counts_kernel()(ea0p, ea1p, dstp, zrows)


_BR = 1024


def _init_body(x_ref, ae_ref, ce_ref, out_ref):
    xb = x_ref[...]
    a = xb[:, 0:1]
    c = xb[:, 1:2]
    oh_a = (lax.broadcasted_iota(jnp.int32, (_BR, 128), 1) == a).astype(jnp.float32)
    oh_c = (lax.broadcasted_iota(jnp.int32, (_BR, 8), 1) == c).astype(jnp.float32)
    h = jnp.dot(oh_a, ae_ref[...], precision=_HIGH) + jnp.dot(
        oh_c, ce_ref[...], precision=_HIGH)
    out_ref[0, :, :] = h[:, :DH]
    out_ref[1, :, :] = h[:, DH:]


def _init_nodes(x_p, atom_emb_p, chir_emb_p):
    return pl.pallas_call(
        _init_body,
        grid=(NPAD // _BR,),
        in_specs=[
            pl.BlockSpec((_BR, 2), lambda i: (i, 0)),
            pl.BlockSpec((128, DP), lambda i: (0, 0)),
            pl.BlockSpec((8, DP), lambda i: (0, 0)),
        ],
        out_specs=pl.BlockSpec((2, _BR, DH), lambda i: (0, i, 0)),
        out_shape=jax.ShapeDtypeStruct((2, NPAD, DH), jnp.float32),
        compiler_params=pltpu.CompilerParams(
            dimension_semantics=("parallel",)),
    )(x_p, atom_emb_p, chir_emb_p)


def _layer_body(relu, p_ref, h_ref, c_ref, ctab_ref, sl_ref, w1_ref, b1_ref,
                w2_ref, b2_ref, sc_ref, be_ref, out_ref):
    aggA = p_ref[0] + h_ref[0]
    aggB = p_ref[1] + h_ref[1]
    agg = jnp.concatenate([aggA, aggB], axis=1)
    cnt = c_ref[0] + c_ref[1]
    agg = agg + jnp.dot(cnt, ctab_ref[...], precision=_HIGH) + sl_ref[...]
    z = jnp.maximum(jnp.dot(agg, w1_ref[...], precision=_HIGH) + b1_ref[...], 0.0)
    z = jnp.dot(z, w2_ref[...], precision=_HIGH) + b2_ref[...]
    z = z * sc_ref[...] + be_ref[...]
    if relu:
        z = jnp.maximum(z, 0.0)
    out_ref[0, :, :] = z[:, :DH]
    out_ref[1, :, :] = z[:, DH:]


def _gin_layer(P, h, C, ctab, sl, w1, b1, w2, b2, scale, beta, relu):
    return pl.pallas_call(
        functools.partial(_layer_body, relu),
        grid=(NPAD // _BR,),
        in_specs=[
            pl.BlockSpec((2, _BR, DH), lambda i: (0, i, 0)),
            pl.BlockSpec((2, _BR, DH), lambda i: (0, i, 0)),
            pl.BlockSpec((2, _BR, CNT), lambda i: (0, i, 0)),
            pl.BlockSpec((CNT, DP), lambda i: (0, 0)),
            pl.BlockSpec((1, DP), lambda i: (0, 0)),
            pl.BlockSpec((DP, HID), lambda i: (0, 0)),
            pl.BlockSpec((1, HID), lambda i: (0, 0)),
            pl.BlockSpec((HID, DP), lambda i: (0, 0)),
            pl.BlockSpec((1, DP), lambda i: (0, 0)),
            pl.BlockSpec((1, DP), lambda i: (0, 0)),
            pl.BlockSpec((1, DP), lambda i: (0, 0)),
        ],
        out_specs=pl.BlockSpec((2, _BR, DH), lambda i: (0, i, 0)),
        out_shape=jax.ShapeDtypeStruct((2, NPAD, DH), jnp.float32),
        compiler_params=pltpu.CompilerParams(
            dimension_semantics=("parallel",)),
    )(P, h, C, ctab, sl, w1, b1, w2, b2, scale, beta)


def _gate_body(h_ref, b_ref, gw_ref, gate_ref, gmax_ref, acc_ref):
    i = pl.program_id(0)
    h = jnp.concatenate([h_ref[0], h_ref[1]], axis=1)
    gate = jnp.dot(h, gw_ref[...], precision=_HIGH)[:, 0:1]
    gate_ref[...] = gate
    seg = lax.broadcasted_iota(jnp.int32, (_BR, G), 1) == b_ref[...]
    bmax = jnp.max(jnp.where(seg, gate, -jnp.inf), axis=0, keepdims=True)

    @pl.when(i == 0)
    def _():
        acc_ref[...] = jnp.full((8, G), -jnp.inf, jnp.float32)

    acc_ref[0:1, :] = jnp.maximum(acc_ref[0:1, :], bmax)
    gmax_ref[...] = jnp.where(
        jnp.isfinite(acc_ref[0:1, :]), acc_ref[0:1, :], 0.0)


def _gate_sweep(h, batch_p, gate_w_p):
    return pl.pallas_call(
        _gate_body,
        grid=(NPAD // _BR,),
        in_specs=[
            pl.BlockSpec((2, _BR, DH), lambda i: (0, i, 0)),
            pl.BlockSpec((_BR, 1), lambda i: (i, 0)),
            pl.BlockSpec((DP, 128), lambda i: (0, 0)),
        ],
        out_specs=[
            pl.BlockSpec((_BR, 1), lambda i: (i, 0)),
            pl.BlockSpec((1, G), lambda i: (0, 0)),
        ],
        out_shape=[
            jax.ShapeDtypeStruct((NPAD, 1), jnp.float32),
            jax.ShapeDtypeStruct((1, G), jnp.float32),
        ],
        scratch_shapes=[pltpu.VMEM((8, G), jnp.float32)],
        compiler_params=pltpu.CompilerParams(
            dimension_semantics=("arbitrary",)),
    )(h, batch_p, gate_w_p)


def _pool_body(h_ref, g_ref, b_ref, gmax_ref, pw_ref, pb_ref, out_ref,
               den_ref, u_ref):
    i = pl.program_id(0)

    @pl.when(i == 0)
    def _():
        den_ref[...] = jnp.zeros_like(den_ref)
        u_ref[...] = jnp.zeros_like(u_ref)

    h = jnp.concatenate([h_ref[0], h_ref[1]], axis=1)
    b = b_ref[...]
    m = (lax.broadcasted_iota(jnp.int32, (_BR, G), 1) == b).astype(jnp.float32)
    gmax_n = jnp.dot(m, gmax_ref[...], precision=_HIGH)
    ex = jnp.where(b < G, jnp.exp(g_ref[...] - gmax_n), 0.0)
    dn = (((0,), (0,)), ((), ()))
    den_ref[...] += lax.dot_general(m, ex, dn, precision=_HIGH)
    u_ref[...] += lax.dot_general(m, ex * h, dn, precision=_HIGH)

    @pl.when(i == pl.num_programs(0) - 1)
    def _():
        pooled = u_ref[...] / (den_ref[...] + 1e-16)
        out_ref[...] = jnp.dot(
            pooled, pw_ref[...], precision=_HIGH) + pb_ref[...]


def _attention_pool(h, gate, batch_p, gmax_t, pred_w_p, pred_b_p):
    return pl.pallas_call(
        _pool_body,
        grid=(NPAD // _BR,),
        in_specs=[
            pl.BlockSpec((2, _BR, DH), lambda i: (0, i, 0)),
            pl.BlockSpec((_BR, 1), lambda i: (i, 0)),
            pl.BlockSpec((_BR, 1), lambda i: (i, 0)),
            pl.BlockSpec((G, 1), lambda i: (0, 0)),
            pl.BlockSpec((DP, DP), lambda i: (0, 0)),
            pl.BlockSpec((1, DP), lambda i: (0, 0)),
        ],
        out_specs=pl.BlockSpec((G, DP), lambda i: (0, 0)),
        out_shape=jax.ShapeDtypeStruct((G, DP), jnp.float32),
        scratch_shapes=[
            pltpu.VMEM((G, 1), jnp.float32),
            pltpu.VMEM((G, DP), jnp.float32),
        ],
        compiler_params=pltpu.CompilerParams(
            dimension_semantics=("arbitrary",)),
    )(h, gate, batch_p, gmax_t, pred_w_p, pred_b_p)


def kernel(x, edge_index, edge_attr, batch, atom_emb, chir_emb, bond_type_emb,
           bond_dir_emb, W1, b1, W2, b2, bn_gamma, bn_beta, gate_w, gate_b,
           pred_w, pred_b):
    eps = 1e-5
    i32 = jnp.int32
    f32 = jnp.float32

    x_p = jnp.zeros((NPAD, 2), i32).at[:N].set(x.astype(i32))
    srcp = jnp.zeros((EPAD,), i32).at[:E].set(edge_index[0].astype(i32))
    dstp = jnp.full((EPAD,), N, i32).at[:E].set(edge_index[1].astype(i32))
    ea0p = jnp.zeros((EPAD,), i32).at[:E].set(edge_attr[:, 0].astype(i32))
    ea1p = jnp.zeros((EPAD,), i32).at[:E].set(edge_attr[:, 1].astype(i32))
    batch_p = jnp.full((NPAD, 1), G, i32).at[:N, 0].set(batch.astype(i32))

    atom_emb_p = jnp.zeros((128, DP), f32).at[:120, :D].set(atom_emb)
    chir_emb_p = jnp.zeros((8, DP), f32).at[:3, :D].set(chir_emb)
    ctabs = (jnp.zeros((L, CNT, DP), f32)
             .at[:, :6, :D].set(bond_type_emb)
             .at[:, 6:9, :D].set(bond_dir_emb))
    sls = jnp.zeros((L, 1, DP), f32).at[:, 0, :D].set(
        bond_type_emb[:, 4, :] + bond_dir_emb[:, 0, :])
    W1p = jnp.zeros((L, DP, HID), f32).at[:, :D, :2 * D].set(W1)
    b1p = jnp.zeros((L, 1, HID), f32).at[:, 0, :2 * D].set(b1)
    W2p = jnp.zeros((L, HID, DP), f32).at[:, :2 * D, :D].set(W2)
    b2p = jnp.zeros((L, 1, DP), f32).at[:, 0, :D].set(b2)
    scales = jnp.zeros((L, 1, DP), f32).at[:, 0, :D].set(
        bn_gamma / jnp.sqrt(1.0 + eps))
    betas = jnp.zeros((L, 1, DP), f32).at[:, 0, :D].set(bn_beta)
    gate_w_p = jnp.zeros((DP, 128), f32).at[:D, 0:1].set(gate_w)
    pred_w_p = jnp.zeros((DP, DP), f32).at[:D, :D].set(pred_w)
    pred_b_p = jnp.zeros((1, DP), f32).at[0, :D].set(pred_b)

    zrows = jnp.zeros((64, DH), f32)
    zrows_c = jnp.zeros((64, CNT), f32)

    h = _init_nodes(x_p, atom_emb_p, chir_emb_p)
    C = _edge_counts(ea0p, ea1p, dstp, zrows_c)
    for l in range(L):
        P = _neighbor_sum(h, srcp, dstp, zrows)
        h = _gin_layer(P, h, C, ctabs[l], sls[l], W1p[l], b1p[l], W2p[l],
                       b2p[l], scales[l], betas[l], relu=(l < L - 1))
    gate, gmax = _gate_sweep(h, batch_p, gate_w_p)
    out = _attention_pool(h, gate, batch_p, gmax.reshape(G, 1), pred_w_p,
                          pred_b_p)
    return out[:, None, :D]


# speedup vs baseline: 6.1761x; 1.0796x over previous
import functools

import jax
import jax.numpy as jnp
from jax import lax
from jax.experimental import pallas as pl
from jax.experimental.pallas import tpu as pltpu
from jax.experimental.pallas import tpu_sc as plsc

N = 10000
NPAD = 10240
D = 300
DP = 320
DH = 160
E = 160000
EPAD = 163840
L = 5
G = 256
HID = 640
CNT = 16
NC, NS = 2, 16
CHUNK = 128
RPT = NPAD // NS

_HIGH = lax.Precision.HIGHEST


@functools.cache
def _sc_mesh():
    return plsc.VectorSubcoreMesh(
        core_axis_name="c", subcore_axis_name="s", num_cores=NC,
        num_subcores=NS)


CH2 = 64


NCHB = 16


def _neigh_body_fn(h, src2, dst2, zrows, out, acc,
                   isb, idb, rows0, rows1,
                   gsem0, gsem1, ssem0, ssem1):
    cid = lax.axis_index("c")
    sid = lax.axis_index("s")

    pltpu.sync_copy(zrows, rows0)
    for i in range(RPT // 64):
        pltpu.async_copy(
            rows0, acc.at[pl.ds(sid * RPT + i * 64, 64)], ssem0)
    for i in range(RPT // 64):
        pltpu.make_async_copy(
            rows0, acc.at[pl.ds(sid * RPT + i * 64, 64)], ssem0).wait()
    plsc.subcore_barrier()

    ept = EPAD // NS
    nblk = ept // (NCHB * CH2)
    rows_ = (rows0, rows1)
    gsem_ = (gsem0, gsem1)
    ssem_ = (ssem0, ssem1)

    def obody(o, carry):
        blk = sid * nblk + o
        pltpu.sync_copy(src2.at[blk], isb)
        pltpu.sync_copy(dst2.at[blk], idb)

        def drain(sl):
            pltpu.make_async_copy(
                rows_[sl], acc.at[idb.at[0]], ssem_[sl]).wait()

        pl.when(o > 0)(lambda: drain(0))
        pltpu.async_copy(h.at[cid].at[isb.at[0]], rows0, gsem0)
        for j in range(NCHB):
            sl = j % 2
            if j + 1 < NCHB:
                if j == 0:
                    pl.when(o > 0)(lambda: drain(1))
                else:
                    drain(1 - sl)
                pltpu.async_copy(
                    h.at[cid].at[isb.at[j + 1]], rows_[1 - sl],
                    gsem_[1 - sl])
            pltpu.make_async_copy(
                h.at[cid].at[isb.at[j]], rows_[sl], gsem_[sl]).wait()
            pltpu.async_copy(
                rows_[sl], acc.at[idb.at[j]], ssem_[sl], add=True)
        return carry

    lax.fori_loop(0, nblk, obody, 0)
    pltpu.make_async_copy(rows0, acc.at[idb.at[0]], ssem0).wait()
    pltpu.make_async_copy(rows1, acc.at[idb.at[1]], ssem1).wait()
    plsc.subcore_barrier()

    nds = RPT // 64
    for i in range(nds):
        buf = rows_[i % 2]
        r = sid * RPT + i * 64
        if i >= 2:
            pltpu.make_async_copy(
                buf, out.at[cid].at[pl.ds(r, 64)], ssem_[i % 2]).wait()
        pltpu.async_copy(acc.at[pl.ds(r, 64)], buf, gsem_[i % 2]).wait()
        pltpu.async_copy(buf, out.at[cid].at[pl.ds(r, 64)], ssem_[i % 2])
    for i in range(nds - 2, nds):
        r = sid * RPT + i * 64
        pltpu.make_async_copy(
            rows_[i % 2], out.at[cid].at[pl.ds(r, 64)], ssem_[i % 2]).wait()


@functools.cache
def _build_neigh_kernel():
    return pl.kernel(
        _neigh_body_fn,
        out_type=jax.ShapeDtypeStruct((2, NPAD, DH), jnp.float32),
        mesh=_sc_mesh(),
        scratch_types=[
            pltpu.VMEM_SHARED((NPAD, DH), jnp.float32),
            pltpu.VMEM((NCHB, CH2), jnp.int32),
            pltpu.VMEM((NCHB, CH2), jnp.int32),
            pltpu.VMEM((CH2, DH), jnp.float32),
            pltpu.VMEM((CH2, DH), jnp.float32),
            pltpu.SemaphoreType.DMA,
            pltpu.SemaphoreType.DMA,
            pltpu.SemaphoreType.DMA,
            pltpu.SemaphoreType.DMA,
        ],
        compiler_params=pltpu.CompilerParams(use_tc_tiling_on_sc=False, needs_layout_passes=False),
    )


def _neighbor_sum(h, srcp, dstp, zrows):
    src2 = srcp.reshape(-1, NCHB, CH2)
    dst2 = dstp.reshape(-1, NCHB, CH2)
    return _build_neigh_kernel()(h, src2, dst2, zrows)


def _counts_body_fn(ea0, ea1, dst, zrows, out,
                    acc, zbuf, rowsc, idx_t, idx_r, idx_d):
    cid = lax.axis_index("c")
    sid = lax.axis_index("s")
    wid = sid * NC + cid

    pltpu.sync_copy(zrows, zbuf)

    def zbody(i, carry):
        pltpu.sync_copy(zbuf, acc.at[pl.ds(sid * RPT + i * 64, 64)])
        return carry

    lax.fori_loop(0, RPT // 64, zbody, 0)
    plsc.subcore_barrier()

    epw = EPAD // (NC * NS)
    ones = jnp.ones((16,), jnp.float32)

    def ebody(c, carry):
        b = wid * epw + c * CHUNK
        pltpu.sync_copy(ea0.at[pl.ds(b, CHUNK)], idx_t)
        pltpu.sync_copy(ea1.at[pl.ds(b, CHUNK)], idx_r)
        pltpu.sync_copy(dst.at[pl.ds(b, CHUNK)], idx_d)
        zero16 = jnp.zeros((16,), jnp.float32)
        for i in range(CHUNK):
            rowsc[i, :] = zero16
        for g in range(CHUNK // 16):
            rid = lax.iota(jnp.int32, 16) + g * 16
            t16 = idx_t[pl.ds(g * 16, 16)]
            plsc.addupdate_scatter(rowsc, [rid, t16], ones)
            r16 = idx_r[pl.ds(g * 16, 16)]
            plsc.addupdate_scatter(rowsc, [rid, r16 + 6], ones)
        pltpu.sync_copy(rowsc, acc.at[idx_d], add=True)
        return carry

    lax.fori_loop(0, epw // CHUNK, ebody, 0)
    plsc.subcore_barrier()

    def dbody(i, carry):
        r = sid * RPT + i * 64
        pltpu.sync_copy(acc.at[pl.ds(r, 64)], zbuf)
        pltpu.sync_copy(zbuf, out.at[cid].at[pl.ds(r, 64)])
        return carry

    lax.fori_loop(0, RPT // 64, dbody, 0)


@functools.cache
def _build_counts_kernel():
    return pl.kernel(
        _counts_body_fn,
        out_type=jax.ShapeDtypeStruct((2, NPAD, CNT), jnp.float32),
        mesh=_sc_mesh(),
        scratch_types=[
            pltpu.VMEM_SHARED((NPAD, CNT), jnp.float32),
            pltpu.VMEM((64, CNT), jnp.float32),
            pltpu.VMEM((CHUNK, CNT), jnp.float32),
            pltpu.VMEM((CHUNK,), jnp.int32),
            pltpu.VMEM((CHUNK,), jnp.int32),
            pltpu.VMEM((CHUNK,), jnp.int32),
        ],
        compiler_params=pltpu.CompilerParams(use_tc_tiling_on_sc=False, needs_layout_passes=False),
    )


def _edge_counts(ea0p, ea1p, dstp, zrows):
    return _build_counts_kernel()(ea0p, ea1p, dstp, zrows)


_BR = 1024


def _init_body(x_ref, ae_ref, ce_ref, out_ref):
    xb = x_ref[...]
    a = xb[:, 0:1]
    c = xb[:, 1:2]
    oh_a = (lax.broadcasted_iota(jnp.int32, (_BR, 128), 1) == a).astype(jnp.float32)
    oh_c = (lax.broadcasted_iota(jnp.int32, (_BR, 8), 1) == c).astype(jnp.float32)
    h = jnp.dot(oh_a, ae_ref[...], precision=_HIGH) + jnp.dot(
        oh_c, ce_ref[...], precision=_HIGH)
    out_ref[0, :, :] = h[:, :DH]
    out_ref[1, :, :] = h[:, DH:]


def _init_nodes(x_p, atom_emb_p, chir_emb_p):
    return pl.pallas_call(
        _init_body,
        grid=(NPAD // _BR,),
        in_specs=[
            pl.BlockSpec((_BR, 2), lambda i: (i, 0)),
            pl.BlockSpec((128, DP), lambda i: (0, 0)),
            pl.BlockSpec((8, DP), lambda i: (0, 0)),
        ],
        out_specs=pl.BlockSpec((2, _BR, DH), lambda i: (0, i, 0)),
        out_shape=jax.ShapeDtypeStruct((2, NPAD, DH), jnp.float32),
        compiler_params=pltpu.CompilerParams(
            dimension_semantics=("parallel",)),
    )(x_p, atom_emb_p, chir_emb_p)


def _layer_body(relu, p_ref, h_ref, c_ref, ctab_ref, sl_ref, w1_ref, b1_ref,
                w2_ref, b2_ref, sc_ref, be_ref, out_ref):
    aggA = p_ref[0] + h_ref[0]
    aggB = p_ref[1] + h_ref[1]
    agg = jnp.concatenate([aggA, aggB], axis=1)
    cnt = c_ref[0] + c_ref[1]
    agg = agg + jnp.dot(cnt, ctab_ref[...], precision=_HIGH) + sl_ref[...]
    z = jnp.maximum(jnp.dot(agg, w1_ref[...], precision=_HIGH) + b1_ref[...], 0.0)
    z = jnp.dot(z, w2_ref[...], precision=_HIGH) + b2_ref[...]
    z = z * sc_ref[...] + be_ref[...]
    if relu:
        z = jnp.maximum(z, 0.0)
    out_ref[0, :, :] = z[:, :DH]
    out_ref[1, :, :] = z[:, DH:]


def _gin_layer(P, h, C, ctab, sl, w1, b1, w2, b2, scale, beta, relu):
    return pl.pallas_call(
        functools.partial(_layer_body, relu),
        grid=(NPAD // _BR,),
        in_specs=[
            pl.BlockSpec((2, _BR, DH), lambda i: (0, i, 0)),
            pl.BlockSpec((2, _BR, DH), lambda i: (0, i, 0)),
            pl.BlockSpec((2, _BR, CNT), lambda i: (0, i, 0)),
            pl.BlockSpec((CNT, DP), lambda i: (0, 0)),
            pl.BlockSpec((1, DP), lambda i: (0, 0)),
            pl.BlockSpec((DP, HID), lambda i: (0, 0)),
            pl.BlockSpec((1, HID), lambda i: (0, 0)),
            pl.BlockSpec((HID, DP), lambda i: (0, 0)),
            pl.BlockSpec((1, DP), lambda i: (0, 0)),
            pl.BlockSpec((1, DP), lambda i: (0, 0)),
            pl.BlockSpec((1, DP), lambda i: (0, 0)),
        ],
        out_specs=pl.BlockSpec((2, _BR, DH), lambda i: (0, i, 0)),
        out_shape=jax.ShapeDtypeStruct((2, NPAD, DH), jnp.float32),
        compiler_params=pltpu.CompilerParams(
            dimension_semantics=("parallel",)),
    )(P, h, C, ctab, sl, w1, b1, w2, b2, scale, beta)


def _gate_body(h_ref, b_ref, gw_ref, gate_ref, gmax_ref, acc_ref):
    i = pl.program_id(0)
    h = jnp.concatenate([h_ref[0], h_ref[1]], axis=1)
    gate = jnp.dot(h, gw_ref[...], precision=_HIGH)[:, 0:1]
    gate_ref[...] = gate
    seg = lax.broadcasted_iota(jnp.int32, (_BR, G), 1) == b_ref[...]
    bmax = jnp.max(jnp.where(seg, gate, -jnp.inf), axis=0, keepdims=True)

    @pl.when(i == 0)
    def _():
        acc_ref[...] = jnp.full((8, G), -jnp.inf, jnp.float32)

    acc_ref[0:1, :] = jnp.maximum(acc_ref[0:1, :], bmax)
    gmax_ref[...] = jnp.where(
        jnp.isfinite(acc_ref[0:1, :]), acc_ref[0:1, :], 0.0)


def _gate_sweep(h, batch_p, gate_w_p):
    return pl.pallas_call(
        _gate_body,
        grid=(NPAD // _BR,),
        in_specs=[
            pl.BlockSpec((2, _BR, DH), lambda i: (0, i, 0)),
            pl.BlockSpec((_BR, 1), lambda i: (i, 0)),
            pl.BlockSpec((DP, 128), lambda i: (0, 0)),
        ],
        out_specs=[
            pl.BlockSpec((_BR, 1), lambda i: (i, 0)),
            pl.BlockSpec((1, G), lambda i: (0, 0)),
        ],
        out_shape=[
            jax.ShapeDtypeStruct((NPAD, 1), jnp.float32),
            jax.ShapeDtypeStruct((1, G), jnp.float32),
        ],
        scratch_shapes=[pltpu.VMEM((8, G), jnp.float32)],
        compiler_params=pltpu.CompilerParams(
            dimension_semantics=("arbitrary",)),
    )(h, batch_p, gate_w_p)


def _pool_body(h_ref, g_ref, b_ref, gmax_ref, pw_ref, pb_ref, out_ref,
               den_ref, u_ref):
    i = pl.program_id(0)

    @pl.when(i == 0)
    def _():
        den_ref[...] = jnp.zeros_like(den_ref)
        u_ref[...] = jnp.zeros_like(u_ref)

    h = jnp.concatenate([h_ref[0], h_ref[1]], axis=1)
    b = b_ref[...]
    m = (lax.broadcasted_iota(jnp.int32, (_BR, G), 1) == b).astype(jnp.float32)
    gmax_n = jnp.dot(m, gmax_ref[...], precision=_HIGH)
    ex = jnp.where(b < G, jnp.exp(g_ref[...] - gmax_n), 0.0)
    dn = (((0,), (0,)), ((), ()))
    den_ref[...] += lax.dot_general(m, ex, dn, precision=_HIGH)
    u_ref[...] += lax.dot_general(m, ex * h, dn, precision=_HIGH)

    @pl.when(i == pl.num_programs(0) - 1)
    def _():
        pooled = u_ref[...] / (den_ref[...] + 1e-16)
        out_ref[...] = jnp.dot(
            pooled, pw_ref[...], precision=_HIGH) + pb_ref[...]


def _attention_pool(h, gate, batch_p, gmax_t, pred_w_p, pred_b_p):
    return pl.pallas_call(
        _pool_body,
        grid=(NPAD // _BR,),
        in_specs=[
            pl.BlockSpec((2, _BR, DH), lambda i: (0, i, 0)),
            pl.BlockSpec((_BR, 1), lambda i: (i, 0)),
            pl.BlockSpec((_BR, 1), lambda i: (i, 0)),
            pl.BlockSpec((G, 1), lambda i: (0, 0)),
            pl.BlockSpec((DP, DP), lambda i: (0, 0)),
            pl.BlockSpec((1, DP), lambda i: (0, 0)),
        ],
        out_specs=pl.BlockSpec((G, DP), lambda i: (0, 0)),
        out_shape=jax.ShapeDtypeStruct((G, DP), jnp.float32),
        scratch_shapes=[
            pltpu.VMEM((G, 1), jnp.float32),
            pltpu.VMEM((G, DP), jnp.float32),
        ],
        compiler_params=pltpu.CompilerParams(
            dimension_semantics=("arbitrary",)),
    )(h, gate, batch_p, gmax_t, pred_w_p, pred_b_p)


def kernel(x, edge_index, edge_attr, batch, atom_emb, chir_emb, bond_type_emb,
           bond_dir_emb, W1, b1, W2, b2, bn_gamma, bn_beta, gate_w, gate_b,
           pred_w, pred_b):
    eps = 1e-5
    i32 = jnp.int32
    f32 = jnp.float32

    x_p = jnp.zeros((NPAD, 2), i32).at[:N].set(x.astype(i32))
    srcp = jnp.zeros((EPAD,), i32).at[:E].set(edge_index[0].astype(i32))
    dstp = jnp.full((EPAD,), N, i32).at[:E].set(edge_index[1].astype(i32))
    ea0p = jnp.zeros((EPAD,), i32).at[:E].set(edge_attr[:, 0].astype(i32))
    ea1p = jnp.zeros((EPAD,), i32).at[:E].set(edge_attr[:, 1].astype(i32))
    batch_p = jnp.full((NPAD, 1), G, i32).at[:N, 0].set(batch.astype(i32))

    atom_emb_p = jnp.zeros((128, DP), f32).at[:120, :D].set(atom_emb)
    chir_emb_p = jnp.zeros((8, DP), f32).at[:3, :D].set(chir_emb)
    ctabs = (jnp.zeros((L, CNT, DP), f32)
             .at[:, :6, :D].set(bond_type_emb)
             .at[:, 6:9, :D].set(bond_dir_emb))
    sls = jnp.zeros((L, 1, DP), f32).at[:, 0, :D].set(
        bond_type_emb[:, 4, :] + bond_dir_emb[:, 0, :])
    W1p = jnp.zeros((L, DP, HID), f32).at[:, :D, :2 * D].set(W1)
    b1p = jnp.zeros((L, 1, HID), f32).at[:, 0, :2 * D].set(b1)
    W2p = jnp.zeros((L, HID, DP), f32).at[:, :2 * D, :D].set(W2)
    b2p = jnp.zeros((L, 1, DP), f32).at[:, 0, :D].set(b2)
    scales = jnp.zeros((L, 1, DP), f32).at[:, 0, :D].set(
        bn_gamma / jnp.sqrt(1.0 + eps))
    betas = jnp.zeros((L, 1, DP), f32).at[:, 0, :D].set(bn_beta)
    gate_w_p = jnp.zeros((DP, 128), f32).at[:D, 0:1].set(gate_w)
    pred_w_p = jnp.zeros((DP, DP), f32).at[:D, :D].set(pred_w)
    pred_b_p = jnp.zeros((1, DP), f32).at[0, :D].set(pred_b)

    zrows = jnp.zeros((64, DH), f32)
    zrows_c = jnp.zeros((64, CNT), f32)

    h = _init_nodes(x_p, atom_emb_p, chir_emb_p)
    C = _edge_counts(ea0p, ea1p, dstp, zrows_c)
    for l in range(L):
        P = _neighbor_sum(h, srcp, dstp, zrows)
        h = _gin_layer(P, h, C, ctabs[l], sls[l], W1p[l], b1p[l], W2p[l],
                       b2p[l], scales[l], betas[l], relu=(l < L - 1))
    gate, gmax = _gate_sweep(h, batch_p, gate_w_p)
    out = _attention_pool(h, gate, batch_p, gmax.reshape(G, 1), pred_w_p,
                          pred_b_p)
    return out[:, None, :D]


# speedup vs baseline: 7.1630x; 1.1598x over previous
import functools

import jax
import jax.numpy as jnp
from jax import lax
from jax.experimental import pallas as pl
from jax.experimental.pallas import tpu as pltpu
from jax.experimental.pallas import tpu_sc as plsc

N = 10000
NPAD = 10240
D = 300
DP = 320
DH = 160
E = 160000
EPAD = 163840
L = 5
G = 256
HID = 640
CNT = 16
NC, NS = 2, 16
CHUNK = 128
RPT = NPAD // NS

_HIGH = lax.Precision.HIGHEST


@functools.cache
def _sc_mesh():
    return plsc.VectorSubcoreMesh(
        core_axis_name="c", subcore_axis_name="s", num_cores=NC,
        num_subcores=NS)


CH2 = 64


NCHB = 16


def _neigh_body_fn(h, src2, dst2, zrows, out, acc,
                   isb, idb, rows0, rows1,
                   gsem0, gsem1, ssem0, ssem1):
    cid = lax.axis_index("c")
    sid = lax.axis_index("s")

    pltpu.sync_copy(zrows, rows0)
    for i in range(RPT // 64):
        pltpu.async_copy(
            rows0, acc.at[pl.ds(sid * RPT + i * 64, 64)], ssem0)
    for i in range(RPT // 64):
        pltpu.make_async_copy(
            rows0, acc.at[pl.ds(sid * RPT + i * 64, 64)], ssem0).wait()
    plsc.subcore_barrier()

    ept = EPAD // NS
    nblk = ept // (NCHB * CH2)
    rows_ = (rows0, rows1)
    gsem_ = (gsem0, gsem1)
    ssem_ = (ssem0, ssem1)

    def obody(o, carry):
        blk = sid * nblk + o
        pltpu.sync_copy(src2.at[blk], isb)
        pltpu.sync_copy(dst2.at[blk], idb)

        def drain(sl):
            pltpu.make_async_copy(
                rows_[sl], acc.at[idb.at[0]], ssem_[sl]).wait()

        pl.when(o > 0)(lambda: drain(0))
        pltpu.async_copy(h.at[cid].at[isb.at[0]], rows0, gsem0)
        for j in range(NCHB):
            sl = j % 2
            if j + 1 < NCHB:
                if j == 0:
                    pl.when(o > 0)(lambda: drain(1))
                else:
                    drain(1 - sl)
                pltpu.async_copy(
                    h.at[cid].at[isb.at[j + 1]], rows_[1 - sl],
                    gsem_[1 - sl])
            pltpu.make_async_copy(
                h.at[cid].at[isb.at[j]], rows_[sl], gsem_[sl]).wait()
            pltpu.async_copy(
                rows_[sl], acc.at[idb.at[j]], ssem_[sl], add=True)
        return carry

    lax.fori_loop(0, nblk, obody, 0)
    pltpu.make_async_copy(rows0, acc.at[idb.at[0]], ssem0).wait()
    pltpu.make_async_copy(rows1, acc.at[idb.at[1]], ssem1).wait()
    plsc.subcore_barrier()

    nds = RPT // 64
    for i in range(nds):
        buf = rows_[i % 2]
        r = sid * RPT + i * 64
        if i >= 2:
            pltpu.make_async_copy(
                buf, out.at[cid].at[pl.ds(r, 64)], ssem_[i % 2]).wait()
        pltpu.async_copy(acc.at[pl.ds(r, 64)], buf, gsem_[i % 2]).wait()
        pltpu.async_copy(buf, out.at[cid].at[pl.ds(r, 64)], ssem_[i % 2])
    for i in range(nds - 2, nds):
        r = sid * RPT + i * 64
        pltpu.make_async_copy(
            rows_[i % 2], out.at[cid].at[pl.ds(r, 64)], ssem_[i % 2]).wait()


@functools.cache
def _build_neigh_kernel():
    return pl.kernel(
        _neigh_body_fn,
        out_type=jax.ShapeDtypeStruct((2, NPAD, DH), jnp.float32),
        mesh=_sc_mesh(),
        scratch_types=[
            pltpu.VMEM_SHARED((NPAD, DH), jnp.float32),
            pltpu.VMEM((NCHB, CH2), jnp.int32),
            pltpu.VMEM((NCHB, CH2), jnp.int32),
            pltpu.VMEM((CH2, DH), jnp.float32),
            pltpu.VMEM((CH2, DH), jnp.float32),
            pltpu.SemaphoreType.DMA,
            pltpu.SemaphoreType.DMA,
            pltpu.SemaphoreType.DMA,
            pltpu.SemaphoreType.DMA,
        ],
        compiler_params=pltpu.CompilerParams(use_tc_tiling_on_sc=False, needs_layout_passes=False),
    )


def _neighbor_sum(h, srcp, dstp, zrows):
    src2 = srcp.reshape(-1, NCHB, CH2)
    dst2 = dstp.reshape(-1, NCHB, CH2)
    return _build_neigh_kernel()(h, src2, dst2, zrows)


def _counts_body_fn(ea0, ea1, dst, zrows, out,
                    acc, zbuf, rowsc, idx_t, idx_r, idx_d):
    cid = lax.axis_index("c")
    sid = lax.axis_index("s")
    wid = sid * NC + cid

    pltpu.sync_copy(zrows, zbuf)

    def zbody(i, carry):
        pltpu.sync_copy(zbuf, acc.at[pl.ds(sid * RPT + i * 64, 64)])
        return carry

    lax.fori_loop(0, RPT // 64, zbody, 0)
    plsc.subcore_barrier()

    epw = EPAD // (NC * NS)
    ones = jnp.ones((16,), jnp.float32)

    def ebody(c, carry):
        b = wid * epw + c * CHUNK
        pltpu.sync_copy(ea0.at[pl.ds(b, CHUNK)], idx_t)
        pltpu.sync_copy(ea1.at[pl.ds(b, CHUNK)], idx_r)
        pltpu.sync_copy(dst.at[pl.ds(b, CHUNK)], idx_d)
        zero16 = jnp.zeros((16,), jnp.float32)
        for i in range(CHUNK):
            rowsc[i, :] = zero16
        for g in range(CHUNK // 16):
            rid = lax.iota(jnp.int32, 16) + g * 16
            t16 = idx_t[pl.ds(g * 16, 16)]
            plsc.addupdate_scatter(rowsc, [rid, t16], ones)
            r16 = idx_r[pl.ds(g * 16, 16)]
            plsc.addupdate_scatter(rowsc, [rid, r16 + 6], ones)
        pltpu.sync_copy(rowsc, acc.at[idx_d], add=True)
        return carry

    lax.fori_loop(0, epw // CHUNK, ebody, 0)
    plsc.subcore_barrier()

    def dbody(i, carry):
        r = sid * RPT + i * 64
        pltpu.sync_copy(acc.at[pl.ds(r, 64)], zbuf)
        pltpu.sync_copy(zbuf, out.at[cid].at[pl.ds(r, 64)])
        return carry

    lax.fori_loop(0, RPT // 64, dbody, 0)


@functools.cache
def _build_counts_kernel():
    return pl.kernel(
        _counts_body_fn,
        out_type=jax.ShapeDtypeStruct((2, NPAD, CNT), jnp.float32),
        mesh=_sc_mesh(),
        scratch_types=[
            pltpu.VMEM_SHARED((NPAD, CNT), jnp.float32),
            pltpu.VMEM((64, CNT), jnp.float32),
            pltpu.VMEM((CHUNK, CNT), jnp.float32),
            pltpu.VMEM((CHUNK,), jnp.int32),
            pltpu.VMEM((CHUNK,), jnp.int32),
            pltpu.VMEM((CHUNK,), jnp.int32),
        ],
        compiler_params=pltpu.CompilerParams(use_tc_tiling_on_sc=False, needs_layout_passes=False),
    )


def _edge_counts(ea0p, ea1p, dstp, zrows):
    return _build_counts_kernel()(ea0p, ea1p, dstp, zrows)


_BR = 1024


def _init_body(x_ref, ae_ref, ce_ref, out_ref):
    xb = x_ref[...]
    a = xb[:, 0:1]
    c = xb[:, 1:2]
    oh_a = (lax.broadcasted_iota(jnp.int32, (_BR, 128), 1) == a).astype(jnp.float32)
    oh_c = (lax.broadcasted_iota(jnp.int32, (_BR, 8), 1) == c).astype(jnp.float32)
    h = jnp.dot(oh_a, ae_ref[...], precision=_HIGH) + jnp.dot(
        oh_c, ce_ref[...], precision=_HIGH)
    out_ref[0, :, :] = h[:, :DH]
    out_ref[1, :, :] = h[:, DH:]


def _init_nodes(x_p, atom_emb_p, chir_emb_p):
    return pl.pallas_call(
        _init_body,
        grid=(NPAD // _BR,),
        in_specs=[
            pl.BlockSpec((_BR, 2), lambda i: (i, 0)),
            pl.BlockSpec((128, DP), lambda i: (0, 0)),
            pl.BlockSpec((8, DP), lambda i: (0, 0)),
        ],
        out_specs=pl.BlockSpec((2, _BR, DH), lambda i: (0, i, 0)),
        out_shape=jax.ShapeDtypeStruct((2, NPAD, DH), jnp.float32),
        compiler_params=pltpu.CompilerParams(
            dimension_semantics=("parallel",)),
    )(x_p, atom_emb_p, chir_emb_p)


def _layer_body(relu, p_ref, h_ref, c_ref, ctab_ref, sl_ref, w1_ref, b1_ref,
                w2_ref, b2_ref, sc_ref, be_ref, out_ref):
    aggA = p_ref[0] + h_ref[0]
    aggB = p_ref[1] + h_ref[1]
    agg = jnp.concatenate([aggA, aggB], axis=1)
    cnt = c_ref[0] + c_ref[1]
    agg = agg + jnp.dot(cnt, ctab_ref[...], precision=_HIGH) + sl_ref[...]
    z = jnp.maximum(
        jnp.dot(agg, w1_ref[...], precision=lax.Precision.DEFAULT)
        + b1_ref[...], 0.0)
    z = jnp.dot(z, w2_ref[...],
                precision=lax.Precision.DEFAULT) + b2_ref[...]
    z = z * sc_ref[...] + be_ref[...]
    if relu:
        z = jnp.maximum(z, 0.0)
    out_ref[0, :, :] = z[:, :DH]
    out_ref[1, :, :] = z[:, DH:]


def _gin_layer(P, h, C, ctab, sl, w1, b1, w2, b2, scale, beta, relu):
    return pl.pallas_call(
        functools.partial(_layer_body, relu),
        grid=(NPAD // _BR,),
        in_specs=[
            pl.BlockSpec((2, _BR, DH), lambda i: (0, i, 0)),
            pl.BlockSpec((2, _BR, DH), lambda i: (0, i, 0)),
            pl.BlockSpec((2, _BR, CNT), lambda i: (0, i, 0)),
            pl.BlockSpec((CNT, DP), lambda i: (0, 0)),
            pl.BlockSpec((1, DP), lambda i: (0, 0)),
            pl.BlockSpec((DP, HID), lambda i: (0, 0)),
            pl.BlockSpec((1, HID), lambda i: (0, 0)),
            pl.BlockSpec((HID, DP), lambda i: (0, 0)),
            pl.BlockSpec((1, DP), lambda i: (0, 0)),
            pl.BlockSpec((1, DP), lambda i: (0, 0)),
            pl.BlockSpec((1, DP), lambda i: (0, 0)),
        ],
        out_specs=pl.BlockSpec((2, _BR, DH), lambda i: (0, i, 0)),
        out_shape=jax.ShapeDtypeStruct((2, NPAD, DH), jnp.float32),
        compiler_params=pltpu.CompilerParams(
            dimension_semantics=("parallel",)),
    )(P, h, C, ctab, sl, w1, b1, w2, b2, scale, beta)


def _gate_body(h_ref, b_ref, gw_ref, gate_ref, gmax_ref, acc_ref):
    i = pl.program_id(0)
    h = jnp.concatenate([h_ref[0], h_ref[1]], axis=1)
    gate = jnp.dot(h, gw_ref[...], precision=_HIGH)[:, 0:1]
    gate_ref[...] = gate
    seg = lax.broadcasted_iota(jnp.int32, (_BR, G), 1) == b_ref[...]
    bmax = jnp.max(jnp.where(seg, gate, -jnp.inf), axis=0, keepdims=True)

    @pl.when(i == 0)
    def _():
        acc_ref[...] = jnp.full((8, G), -jnp.inf, jnp.float32)

    acc_ref[0:1, :] = jnp.maximum(acc_ref[0:1, :], bmax)
    gmax_ref[...] = jnp.where(
        jnp.isfinite(acc_ref[0:1, :]), acc_ref[0:1, :], 0.0)


def _gate_sweep(h, batch_p, gate_w_p):
    return pl.pallas_call(
        _gate_body,
        grid=(NPAD // _BR,),
        in_specs=[
            pl.BlockSpec((2, _BR, DH), lambda i: (0, i, 0)),
            pl.BlockSpec((_BR, 1), lambda i: (i, 0)),
            pl.BlockSpec((DP, 128), lambda i: (0, 0)),
        ],
        out_specs=[
            pl.BlockSpec((_BR, 1), lambda i: (i, 0)),
            pl.BlockSpec((1, G), lambda i: (0, 0)),
        ],
        out_shape=[
            jax.ShapeDtypeStruct((NPAD, 1), jnp.float32),
            jax.ShapeDtypeStruct((1, G), jnp.float32),
        ],
        scratch_shapes=[pltpu.VMEM((8, G), jnp.float32)],
        compiler_params=pltpu.CompilerParams(
            dimension_semantics=("arbitrary",)),
    )(h, batch_p, gate_w_p)


def _pool_body(h_ref, g_ref, b_ref, gmax_ref, pw_ref, pb_ref, out_ref,
               den_ref, u_ref):
    i = pl.program_id(0)

    @pl.when(i == 0)
    def _():
        den_ref[...] = jnp.zeros_like(den_ref)
        u_ref[...] = jnp.zeros_like(u_ref)

    h = jnp.concatenate([h_ref[0], h_ref[1]], axis=1)
    b = b_ref[...]
    m = (lax.broadcasted_iota(jnp.int32, (_BR, G), 1) == b).astype(jnp.float32)
    gmax_n = jnp.dot(m, gmax_ref[...], precision=_HIGH)
    ex = jnp.where(b < G, jnp.exp(g_ref[...] - gmax_n), 0.0)
    dn = (((0,), (0,)), ((), ()))
    den_ref[...] += lax.dot_general(m, ex, dn, precision=_HIGH)
    u_ref[...] += lax.dot_general(m, ex * h, dn, precision=_HIGH)

    @pl.when(i == pl.num_programs(0) - 1)
    def _():
        pooled = u_ref[...] / (den_ref[...] + 1e-16)
        out_ref[...] = jnp.dot(
            pooled, pw_ref[...], precision=_HIGH) + pb_ref[...]


def _attention_pool(h, gate, batch_p, gmax_t, pred_w_p, pred_b_p):
    return pl.pallas_call(
        _pool_body,
        grid=(NPAD // _BR,),
        in_specs=[
            pl.BlockSpec((2, _BR, DH), lambda i: (0, i, 0)),
            pl.BlockSpec((_BR, 1), lambda i: (i, 0)),
            pl.BlockSpec((_BR, 1), lambda i: (i, 0)),
            pl.BlockSpec((G, 1), lambda i: (0, 0)),
            pl.BlockSpec((DP, DP), lambda i: (0, 0)),
            pl.BlockSpec((1, DP), lambda i: (0, 0)),
        ],
        out_specs=pl.BlockSpec((G, DP), lambda i: (0, 0)),
        out_shape=jax.ShapeDtypeStruct((G, DP), jnp.float32),
        scratch_shapes=[
            pltpu.VMEM((G, 1), jnp.float32),
            pltpu.VMEM((G, DP), jnp.float32),
        ],
        compiler_params=pltpu.CompilerParams(
            dimension_semantics=("arbitrary",)),
    )(h, gate, batch_p, gmax_t, pred_w_p, pred_b_p)


def kernel(x, edge_index, edge_attr, batch, atom_emb, chir_emb, bond_type_emb,
           bond_dir_emb, W1, b1, W2, b2, bn_gamma, bn_beta, gate_w, gate_b,
           pred_w, pred_b):
    eps = 1e-5
    i32 = jnp.int32
    f32 = jnp.float32

    x_p = jnp.zeros((NPAD, 2), i32).at[:N].set(x.astype(i32))
    srcp = jnp.zeros((EPAD,), i32).at[:E].set(edge_index[0].astype(i32))
    dstp = jnp.full((EPAD,), N, i32).at[:E].set(edge_index[1].astype(i32))
    ea0p = jnp.zeros((EPAD,), i32).at[:E].set(edge_attr[:, 0].astype(i32))
    ea1p = jnp.zeros((EPAD,), i32).at[:E].set(edge_attr[:, 1].astype(i32))
    batch_p = jnp.full((NPAD, 1), G, i32).at[:N, 0].set(batch.astype(i32))

    atom_emb_p = jnp.zeros((128, DP), f32).at[:120, :D].set(atom_emb)
    chir_emb_p = jnp.zeros((8, DP), f32).at[:3, :D].set(chir_emb)
    ctabs = (jnp.zeros((L, CNT, DP), f32)
             .at[:, :6, :D].set(bond_type_emb)
             .at[:, 6:9, :D].set(bond_dir_emb))
    sls = jnp.zeros((L, 1, DP), f32).at[:, 0, :D].set(
        bond_type_emb[:, 4, :] + bond_dir_emb[:, 0, :])
    W1p = jnp.zeros((L, DP, HID), f32).at[:, :D, :2 * D].set(W1)
    b1p = jnp.zeros((L, 1, HID), f32).at[:, 0, :2 * D].set(b1)
    W2p = jnp.zeros((L, HID, DP), f32).at[:, :2 * D, :D].set(W2)
    b2p = jnp.zeros((L, 1, DP), f32).at[:, 0, :D].set(b2)
    scales = jnp.zeros((L, 1, DP), f32).at[:, 0, :D].set(
        bn_gamma / jnp.sqrt(1.0 + eps))
    betas = jnp.zeros((L, 1, DP), f32).at[:, 0, :D].set(bn_beta)
    gate_w_p = jnp.zeros((DP, 128), f32).at[:D, 0:1].set(gate_w)
    pred_w_p = jnp.zeros((DP, DP), f32).at[:D, :D].set(pred_w)
    pred_b_p = jnp.zeros((1, DP), f32).at[0, :D].set(pred_b)

    zrows = jnp.zeros((64, DH), f32)
    zrows_c = jnp.zeros((64, CNT), f32)

    h = _init_nodes(x_p, atom_emb_p, chir_emb_p)
    C = _edge_counts(ea0p, ea1p, dstp, zrows_c)
    for l in range(L):
        P = _neighbor_sum(h, srcp, dstp, zrows)
        h = _gin_layer(P, h, C, ctabs[l], sls[l], W1p[l], b1p[l], W2p[l],
                       b2p[l], scales[l], betas[l], relu=(l < L - 1))
    gate, gmax = _gate_sweep(h, batch_p, gate_w_p)
    out = _attention_pool(h, gate, batch_p, gmax.reshape(G, 1), pred_w_p,
                          pred_b_p)
    return out[:, None, :D]


# speedup vs baseline: 7.3476x; 1.0258x over previous
import functools

import jax
import jax.numpy as jnp
from jax import lax
from jax.experimental import pallas as pl
from jax.experimental.pallas import tpu as pltpu
from jax.experimental.pallas import tpu_sc as plsc

N = 10000
NPAD = 10240
D = 300
DP = 320
DH = 160
E = 160000
EPAD = 163840
L = 5
G = 256
HID = 640
CNT = 16
NC, NS = 2, 16
CHUNK = 128
RPT = NPAD // NS

_HIGH = lax.Precision.HIGHEST


@functools.cache
def _sc_mesh():
    return plsc.VectorSubcoreMesh(
        core_axis_name="c", subcore_axis_name="s", num_cores=NC,
        num_subcores=NS)


CH2 = 64


NCHB = 16


def _neigh_body_fn(h, src2, dst2, zrows, out, acc,
                   isb, idb, rows0, rows1,
                   gsem0, gsem1, ssem0, ssem1, isem0, isem1):
    cid = lax.axis_index("c")
    sid = lax.axis_index("s")

    pltpu.sync_copy(zrows, rows0)
    for i in range(RPT // 64):
        pltpu.async_copy(
            rows0, acc.at[pl.ds(sid * RPT + i * 64, 64)], ssem0)
    for i in range(RPT // 64):
        pltpu.make_async_copy(
            rows0, acc.at[pl.ds(sid * RPT + i * 64, 64)], ssem0).wait()
    plsc.subcore_barrier()

    ept = EPAD // NS
    nblk = ept // (NCHB * CH2)
    rows_ = (rows0, rows1)
    gsem_ = (gsem0, gsem1)
    ssem_ = (ssem0, ssem1)

    pltpu.sync_copy(src2.at[sid * nblk], isb.at[0])
    pltpu.sync_copy(dst2.at[sid * nblk], idb.at[0])

    def obody(o, carry):
        blk = sid * nblk + o
        ib = o % 2
        isbo = isb.at[ib]
        idbo = idb.at[ib]

        def prefetch_idx():
            pltpu.async_copy(src2.at[blk + 1], isb.at[1 - ib], isem0)
            pltpu.async_copy(dst2.at[blk + 1], idb.at[1 - ib], isem1)

        pl.when(o + 1 < nblk)(prefetch_idx)

        def drain(sl):
            pltpu.make_async_copy(
                rows_[sl], acc.at[idbo.at[0]], ssem_[sl]).wait()

        pl.when(o > 0)(lambda: drain(0))
        pltpu.async_copy(h.at[cid].at[isbo.at[0]], rows0, gsem0)
        for j in range(NCHB):
            sl = j % 2
            if j + 1 < NCHB:
                if j == 0:
                    pl.when(o > 0)(lambda: drain(1))
                else:
                    drain(1 - sl)
                pltpu.async_copy(
                    h.at[cid].at[isbo.at[j + 1]], rows_[1 - sl],
                    gsem_[1 - sl])
            pltpu.make_async_copy(
                h.at[cid].at[isbo.at[j]], rows_[sl], gsem_[sl]).wait()
            pltpu.async_copy(
                rows_[sl], acc.at[idbo.at[j]], ssem_[sl], add=True)

        def wait_idx():
            pltpu.make_async_copy(
                src2.at[blk + 1], isb.at[1 - ib], isem0).wait()
            pltpu.make_async_copy(
                dst2.at[blk + 1], idb.at[1 - ib], isem1).wait()

        pl.when(o + 1 < nblk)(wait_idx)
        return carry

    lax.fori_loop(0, nblk, obody, 0)
    pltpu.make_async_copy(rows0, acc.at[idb.at[0].at[0]], ssem0).wait()
    pltpu.make_async_copy(rows1, acc.at[idb.at[0].at[1]], ssem1).wait()
    plsc.subcore_barrier()

    nds = RPT // 64
    for i in range(nds):
        buf = rows_[i % 2]
        r = sid * RPT + i * 64
        if i >= 2:
            pltpu.make_async_copy(
                buf, out.at[cid].at[pl.ds(r, 64)], ssem_[i % 2]).wait()
        pltpu.async_copy(acc.at[pl.ds(r, 64)], buf, gsem_[i % 2]).wait()
        pltpu.async_copy(buf, out.at[cid].at[pl.ds(r, 64)], ssem_[i % 2])
    for i in range(nds - 2, nds):
        r = sid * RPT + i * 64
        pltpu.make_async_copy(
            rows_[i % 2], out.at[cid].at[pl.ds(r, 64)], ssem_[i % 2]).wait()


@functools.cache
def _build_neigh_kernel():
    return pl.kernel(
        _neigh_body_fn,
        out_type=jax.ShapeDtypeStruct((2, NPAD, DH), jnp.float32),
        mesh=_sc_mesh(),
        scratch_types=[
            pltpu.VMEM_SHARED((NPAD, DH), jnp.float32),
            pltpu.VMEM((2, NCHB, CH2), jnp.int32),
            pltpu.VMEM((2, NCHB, CH2), jnp.int32),
            pltpu.VMEM((CH2, DH), jnp.float32),
            pltpu.VMEM((CH2, DH), jnp.float32),
            pltpu.SemaphoreType.DMA,
            pltpu.SemaphoreType.DMA,
            pltpu.SemaphoreType.DMA,
            pltpu.SemaphoreType.DMA,
            pltpu.SemaphoreType.DMA,
            pltpu.SemaphoreType.DMA,
        ],
        compiler_params=pltpu.CompilerParams(use_tc_tiling_on_sc=False, needs_layout_passes=False),
    )


def _neighbor_sum(h, srcp, dstp, zrows):
    src2 = srcp.reshape(-1, NCHB, CH2)
    dst2 = dstp.reshape(-1, NCHB, CH2)
    return _build_neigh_kernel()(h, src2, dst2, zrows)


def _counts_body_fn(ea0, ea1, dst, zrows, out,
                    acc, zbuf, rowsc, idx_t, idx_r, idx_d):
    cid = lax.axis_index("c")
    sid = lax.axis_index("s")
    wid = sid * NC + cid

    pltpu.sync_copy(zrows, zbuf)

    def zbody(i, carry):
        pltpu.sync_copy(zbuf, acc.at[pl.ds(sid * RPT + i * 64, 64)])
        return carry

    lax.fori_loop(0, RPT // 64, zbody, 0)
    plsc.subcore_barrier()

    epw = EPAD // (NC * NS)
    ones = jnp.ones((16,), jnp.float32)

    def ebody(c, carry):
        b = wid * epw + c * CHUNK
        pltpu.sync_copy(ea0.at[pl.ds(b, CHUNK)], idx_t)
        pltpu.sync_copy(ea1.at[pl.ds(b, CHUNK)], idx_r)
        pltpu.sync_copy(dst.at[pl.ds(b, CHUNK)], idx_d)
        zero16 = jnp.zeros((16,), jnp.float32)
        for i in range(CHUNK):
            rowsc[i, :] = zero16
        for g in range(CHUNK // 16):
            rid = lax.iota(jnp.int32, 16) + g * 16
            t16 = idx_t[pl.ds(g * 16, 16)]
            plsc.addupdate_scatter(rowsc, [rid, t16], ones)
            r16 = idx_r[pl.ds(g * 16, 16)]
            plsc.addupdate_scatter(rowsc, [rid, r16 + 6], ones)
        pltpu.sync_copy(rowsc, acc.at[idx_d], add=True)
        return carry

    lax.fori_loop(0, epw // CHUNK, ebody, 0)
    plsc.subcore_barrier()

    def dbody(i, carry):
        r = sid * RPT + i * 64
        pltpu.sync_copy(acc.at[pl.ds(r, 64)], zbuf)
        pltpu.sync_copy(zbuf, out.at[cid].at[pl.ds(r, 64)])
        return carry

    lax.fori_loop(0, RPT // 64, dbody, 0)


@functools.cache
def _build_counts_kernel():
    return pl.kernel(
        _counts_body_fn,
        out_type=jax.ShapeDtypeStruct((2, NPAD, CNT), jnp.float32),
        mesh=_sc_mesh(),
        scratch_types=[
            pltpu.VMEM_SHARED((NPAD, CNT), jnp.float32),
            pltpu.VMEM((64, CNT), jnp.float32),
            pltpu.VMEM((CHUNK, CNT), jnp.float32),
            pltpu.VMEM((CHUNK,), jnp.int32),
            pltpu.VMEM((CHUNK,), jnp.int32),
            pltpu.VMEM((CHUNK,), jnp.int32),
        ],
        compiler_params=pltpu.CompilerParams(use_tc_tiling_on_sc=False, needs_layout_passes=False),
    )


def _edge_counts(ea0p, ea1p, dstp, zrows):
    return _build_counts_kernel()(ea0p, ea1p, dstp, zrows)


_BR = 1024


def _init_body(x_ref, ae_ref, ce_ref, out_ref):
    xb = x_ref[...]
    a = xb[:, 0:1]
    c = xb[:, 1:2]
    oh_a = (lax.broadcasted_iota(jnp.int32, (_BR, 128), 1) == a).astype(jnp.float32)
    oh_c = (lax.broadcasted_iota(jnp.int32, (_BR, 8), 1) == c).astype(jnp.float32)
    h = jnp.dot(oh_a, ae_ref[...], precision=_HIGH) + jnp.dot(
        oh_c, ce_ref[...], precision=_HIGH)
    out_ref[0, :, :] = h[:, :DH]
    out_ref[1, :, :] = h[:, DH:]


def _init_nodes(x_p, atom_emb_p, chir_emb_p):
    return pl.pallas_call(
        _init_body,
        grid=(NPAD // _BR,),
        in_specs=[
            pl.BlockSpec((_BR, 2), lambda i: (i, 0)),
            pl.BlockSpec((128, DP), lambda i: (0, 0)),
            pl.BlockSpec((8, DP), lambda i: (0, 0)),
        ],
        out_specs=pl.BlockSpec((2, _BR, DH), lambda i: (0, i, 0)),
        out_shape=jax.ShapeDtypeStruct((2, NPAD, DH), jnp.float32),
        compiler_params=pltpu.CompilerParams(
            dimension_semantics=("parallel",)),
    )(x_p, atom_emb_p, chir_emb_p)


def _layer_body(relu, p_ref, h_ref, c_ref, ctab_ref, sl_ref, w1_ref, b1_ref,
                w2_ref, b2_ref, sc_ref, be_ref, out_ref):
    aggA = p_ref[0] + h_ref[0]
    aggB = p_ref[1] + h_ref[1]
    agg = jnp.concatenate([aggA, aggB], axis=1)
    cnt = c_ref[0] + c_ref[1]
    agg = agg + jnp.dot(cnt, ctab_ref[...], precision=_HIGH) + sl_ref[...]
    z = jnp.maximum(
        jnp.dot(agg, w1_ref[...], precision=lax.Precision.DEFAULT)
        + b1_ref[...], 0.0)
    z = jnp.dot(z, w2_ref[...],
                precision=lax.Precision.DEFAULT) + b2_ref[...]
    z = z * sc_ref[...] + be_ref[...]
    if relu:
        z = jnp.maximum(z, 0.0)
    out_ref[0, :, :] = z[:, :DH]
    out_ref[1, :, :] = z[:, DH:]


def _gin_layer(P, h, C, ctab, sl, w1, b1, w2, b2, scale, beta, relu):
    return pl.pallas_call(
        functools.partial(_layer_body, relu),
        grid=(NPAD // _BR,),
        in_specs=[
            pl.BlockSpec((2, _BR, DH), lambda i: (0, i, 0)),
            pl.BlockSpec((2, _BR, DH), lambda i: (0, i, 0)),
            pl.BlockSpec((2, _BR, CNT), lambda i: (0, i, 0)),
            pl.BlockSpec((CNT, DP), lambda i: (0, 0)),
            pl.BlockSpec((1, DP), lambda i: (0, 0)),
            pl.BlockSpec((DP, HID), lambda i: (0, 0)),
            pl.BlockSpec((1, HID), lambda i: (0, 0)),
            pl.BlockSpec((HID, DP), lambda i: (0, 0)),
            pl.BlockSpec((1, DP), lambda i: (0, 0)),
            pl.BlockSpec((1, DP), lambda i: (0, 0)),
            pl.BlockSpec((1, DP), lambda i: (0, 0)),
        ],
        out_specs=pl.BlockSpec((2, _BR, DH), lambda i: (0, i, 0)),
        out_shape=jax.ShapeDtypeStruct((2, NPAD, DH), jnp.float32),
        compiler_params=pltpu.CompilerParams(
            dimension_semantics=("parallel",)),
    )(P, h, C, ctab, sl, w1, b1, w2, b2, scale, beta)


def _gate_body(h_ref, b_ref, gw_ref, gate_ref, gmax_ref, acc_ref):
    i = pl.program_id(0)
    h = jnp.concatenate([h_ref[0], h_ref[1]], axis=1)
    gate = jnp.dot(h, gw_ref[...], precision=_HIGH)[:, 0:1]
    gate_ref[...] = gate
    seg = lax.broadcasted_iota(jnp.int32, (_BR, G), 1) == b_ref[...]
    bmax = jnp.max(jnp.where(seg, gate, -jnp.inf), axis=0, keepdims=True)

    @pl.when(i == 0)
    def _():
        acc_ref[...] = jnp.full((8, G), -jnp.inf, jnp.float32)

    acc_ref[0:1, :] = jnp.maximum(acc_ref[0:1, :], bmax)
    gmax_ref[...] = jnp.where(
        jnp.isfinite(acc_ref[0:1, :]), acc_ref[0:1, :], 0.0)


def _gate_sweep(h, batch_p, gate_w_p):
    return pl.pallas_call(
        _gate_body,
        grid=(NPAD // _BR,),
        in_specs=[
            pl.BlockSpec((2, _BR, DH), lambda i: (0, i, 0)),
            pl.BlockSpec((_BR, 1), lambda i: (i, 0)),
            pl.BlockSpec((DP, 128), lambda i: (0, 0)),
        ],
        out_specs=[
            pl.BlockSpec((_BR, 1), lambda i: (i, 0)),
            pl.BlockSpec((1, G), lambda i: (0, 0)),
        ],
        out_shape=[
            jax.ShapeDtypeStruct((NPAD, 1), jnp.float32),
            jax.ShapeDtypeStruct((1, G), jnp.float32),
        ],
        scratch_shapes=[pltpu.VMEM((8, G), jnp.float32)],
        compiler_params=pltpu.CompilerParams(
            dimension_semantics=("arbitrary",)),
    )(h, batch_p, gate_w_p)


def _pool_body(h_ref, g_ref, b_ref, gmax_ref, pw_ref, pb_ref, out_ref,
               den_ref, u_ref):
    i = pl.program_id(0)

    @pl.when(i == 0)
    def _():
        den_ref[...] = jnp.zeros_like(den_ref)
        u_ref[...] = jnp.zeros_like(u_ref)

    h = jnp.concatenate([h_ref[0], h_ref[1]], axis=1)
    b = b_ref[...]
    m = (lax.broadcasted_iota(jnp.int32, (_BR, G), 1) == b).astype(jnp.float32)
    gmax_n = jnp.dot(m, gmax_ref[...], precision=_HIGH)
    ex = jnp.where(b < G, jnp.exp(g_ref[...] - gmax_n), 0.0)
    dn = (((0,), (0,)), ((), ()))
    den_ref[...] += lax.dot_general(m, ex, dn, precision=_HIGH)
    u_ref[...] += lax.dot_general(m, ex * h, dn, precision=_HIGH)

    @pl.when(i == pl.num_programs(0) - 1)
    def _():
        pooled = u_ref[...] / (den_ref[...] + 1e-16)
        out_ref[...] = jnp.dot(
            pooled, pw_ref[...], precision=_HIGH) + pb_ref[...]


def _attention_pool(h, gate, batch_p, gmax_t, pred_w_p, pred_b_p):
    return pl.pallas_call(
        _pool_body,
        grid=(NPAD // _BR,),
        in_specs=[
            pl.BlockSpec((2, _BR, DH), lambda i: (0, i, 0)),
            pl.BlockSpec((_BR, 1), lambda i: (i, 0)),
            pl.BlockSpec((_BR, 1), lambda i: (i, 0)),
            pl.BlockSpec((G, 1), lambda i: (0, 0)),
            pl.BlockSpec((DP, DP), lambda i: (0, 0)),
            pl.BlockSpec((1, DP), lambda i: (0, 0)),
        ],
        out_specs=pl.BlockSpec((G, DP), lambda i: (0, 0)),
        out_shape=jax.ShapeDtypeStruct((G, DP), jnp.float32),
        scratch_shapes=[
            pltpu.VMEM((G, 1), jnp.float32),
            pltpu.VMEM((G, DP), jnp.float32),
        ],
        compiler_params=pltpu.CompilerParams(
            dimension_semantics=("arbitrary",)),
    )(h, gate, batch_p, gmax_t, pred_w_p, pred_b_p)


def kernel(x, edge_index, edge_attr, batch, atom_emb, chir_emb, bond_type_emb,
           bond_dir_emb, W1, b1, W2, b2, bn_gamma, bn_beta, gate_w, gate_b,
           pred_w, pred_b):
    eps = 1e-5
    i32 = jnp.int32
    f32 = jnp.float32

    x_p = jnp.zeros((NPAD, 2), i32).at[:N].set(x.astype(i32))
    srcp = jnp.zeros((EPAD,), i32).at[:E].set(edge_index[0].astype(i32))
    dstp = jnp.full((EPAD,), N, i32).at[:E].set(edge_index[1].astype(i32))
    ea0p = jnp.zeros((EPAD,), i32).at[:E].set(edge_attr[:, 0].astype(i32))
    ea1p = jnp.zeros((EPAD,), i32).at[:E].set(edge_attr[:, 1].astype(i32))
    batch_p = jnp.full((NPAD, 1), G, i32).at[:N, 0].set(batch.astype(i32))

    atom_emb_p = jnp.zeros((128, DP), f32).at[:120, :D].set(atom_emb)
    chir_emb_p = jnp.zeros((8, DP), f32).at[:3, :D].set(chir_emb)
    ctabs = (jnp.zeros((L, CNT, DP), f32)
             .at[:, :6, :D].set(bond_type_emb)
             .at[:, 6:9, :D].set(bond_dir_emb))
    sls = jnp.zeros((L, 1, DP), f32).at[:, 0, :D].set(
        bond_type_emb[:, 4, :] + bond_dir_emb[:, 0, :])
    W1p = jnp.zeros((L, DP, HID), f32).at[:, :D, :2 * D].set(W1)
    b1p = jnp.zeros((L, 1, HID), f32).at[:, 0, :2 * D].set(b1)
    W2p = jnp.zeros((L, HID, DP), f32).at[:, :2 * D, :D].set(W2)
    b2p = jnp.zeros((L, 1, DP), f32).at[:, 0, :D].set(b2)
    scales = jnp.zeros((L, 1, DP), f32).at[:, 0, :D].set(
        bn_gamma / jnp.sqrt(1.0 + eps))
    betas = jnp.zeros((L, 1, DP), f32).at[:, 0, :D].set(bn_beta)
    gate_w_p = jnp.zeros((DP, 128), f32).at[:D, 0:1].set(gate_w)
    pred_w_p = jnp.zeros((DP, DP), f32).at[:D, :D].set(pred_w)
    pred_b_p = jnp.zeros((1, DP), f32).at[0, :D].set(pred_b)

    zrows = jnp.zeros((64, DH), f32)
    zrows_c = jnp.zeros((64, CNT), f32)

    h = _init_nodes(x_p, atom_emb_p, chir_emb_p)
    C = _edge_counts(ea0p, ea1p, dstp, zrows_c)
    for l in range(L):
        P = _neighbor_sum(h, srcp, dstp, zrows)
        h = _gin_layer(P, h, C, ctabs[l], sls[l], W1p[l], b1p[l], W2p[l],
                       b2p[l], scales[l], betas[l], relu=(l < L - 1))
    gate, gmax = _gate_sweep(h, batch_p, gate_w_p)
    out = _attention_pool(h, gate, batch_p, gmax.reshape(G, 1), pred_w_p,
                          pred_b_p)
    return out[:, None, :D]
